# Initial kernel scaffold; baseline (speedup 1.0000x reference)
import jax
import jax.numpy as jnp
from jax.experimental import pallas as pl


def kernel(node_feats, edge_index, Wg1, bg1, Wg2, bg2, Wea, bea, Wna, bna, Wnir, bnir, Wm1, bm1, Wm2, bm2, Wn1, gamma_n, beta_n, Wn2, We1, gamma_e, beta_e, We2):
    raise NotImplementedError("write your pallas kernel here")


# speedup vs baseline: 5.0962x; 5.0962x over previous
import functools

import jax
import jax.numpy as jnp
from jax import lax
from jax.experimental import pallas as pl
from jax.experimental.pallas import tpu as pltpu
from jax.experimental.pallas import tpu_sc as plsc

N = 10000
E = 320000
NC = 2
NS = 16
NW = NC * NS
EW = E // NW
CH = 80
NCHUNK = EW // CH
NPAD = 10240
SLAB = NPAD // NS

RB = 400
NB = N // RB
EBLK = 512
NEB = E // EBLK

_F32 = jnp.float32
_MASK_HI = -65536


def _sds(shape, dtype=_F32):
    return jax.ShapeDtypeStruct(shape, dtype)


def _sigmoid(x):
    return 1.0 / (1.0 + jnp.exp(-x))


def _bf16_bits(x):
    b = lax.bitcast_convert_type(x, jnp.int32)
    return b + 0x7FFF + (lax.shift_right_logical(b, 16) & 1)


def _pack2(hi, lo):
    return (_bf16_bits(hi) & _MASK_HI) | lax.shift_right_logical(_bf16_bits(lo), 16)


def _unpack_hi(w):
    return lax.bitcast_convert_type(w & _MASK_HI, _F32)


def _unpack_lo(w):
    return lax.bitcast_convert_type(jnp.left_shift(w, 16), _F32)


def _mesh():
    return plsc.VectorSubcoreMesh(
        core_axis_name="c", subcore_axis_name="s", num_cores=NC, num_subcores=NS)


def _sc_segment_sum(table, gather_idx, scatter_idx, zeros, width):
    @functools.partial(
        pl.kernel,
        out_type=_sds((NC, NPAD, width)),
        mesh=_mesh(),
        scratch_types=[
            pltpu.VMEM((CH,), jnp.int32),
            pltpu.VMEM((CH,), jnp.int32),
            pltpu.VMEM((CH, width), _F32),
            pltpu.VMEM_SHARED((NPAD, width), _F32),
            pltpu.SemaphoreType.DMA,
        ],
    )
    def body(tab_hbm, s_hbm, o_hbm, z_hbm, out_hbm, sidx, oidx, rows, acc, sem):
        cid = lax.axis_index("c")
        sid = lax.axis_index("s")
        wid = cid * NS + sid
        pltpu.sync_copy(z_hbm, acc.at[pl.ds(sid * SLAB, SLAB), :])
        plsc.subcore_barrier()
        base = wid * EW

        def step(i, carry):
            off = base + i * CH
            pltpu.sync_copy(s_hbm.at[pl.ds(off, CH)], sidx)
            pltpu.sync_copy(o_hbm.at[pl.ds(off, CH)], oidx)
            pltpu.async_copy(tab_hbm.at[sidx], rows, sem).wait()
            pltpu.sync_copy(rows, acc.at[oidx], add=True)
            return carry

        lax.fori_loop(0, NCHUNK, step, 0)
        plsc.subcore_barrier()
        sl = pl.ds(sid * SLAB, SLAB)
        pltpu.sync_copy(acc.at[sl, :], out_hbm.at[cid, sl, :])

    return body(table, gather_idx, scatter_idx, zeros)


def _sc_edge_gather(upi, vqi, subj, obj):
    @functools.partial(
        pl.kernel,
        out_type=[_sds((E, 128), jnp.int32), _sds((E, 128), jnp.int32)],
        mesh=_mesh(),
        scratch_types=[
            pltpu.VMEM((CH,), jnp.int32),
            pltpu.VMEM((CH,), jnp.int32),
            pltpu.VMEM((CH, 128), jnp.int32),
            pltpu.VMEM((CH, 128), jnp.int32),
            pltpu.SemaphoreType.DMA,
            pltpu.SemaphoreType.DMA,
        ],
    )
    def body(up_hbm, vq_hbm, s_hbm, o_hbm, g1_hbm, g2_hbm,
             sidx, oidx, r1, r2, sem1, sem2):
        cid = lax.axis_index("c")
        sid = lax.axis_index("s")
        wid = cid * NS + sid
        base = wid * EW

        def step(i, carry):
            off = base + i * CH
            pltpu.sync_copy(s_hbm.at[pl.ds(off, CH)], sidx)
            pltpu.sync_copy(o_hbm.at[pl.ds(off, CH)], oidx)
            d1 = pltpu.async_copy(up_hbm.at[sidx], r1, sem1)
            d2 = pltpu.async_copy(vq_hbm.at[oidx], r2, sem2)
            d1.wait()
            pltpu.sync_copy(r1, g1_hbm.at[pl.ds(off, CH), :])
            d2.wait()
            pltpu.sync_copy(r2, g2_hbm.at[pl.ds(off, CH), :])
            return carry

        lax.fori_loop(0, NCHUNK, step, 0)

    return body(upi, vqi, subj, obj)


def _dot(a, b):
    return jnp.dot(a, b, preferred_element_type=_F32)


def _tc_proj(x, wa, wb, wg1, wu, wv):
    def body(x_r, wa_r, wb_r, wg_r, wu_r, wv_r, aaug_o, baug_o, h1_o, u_o, v_o):
        xb = x_r[...]
        ones = jnp.ones((RB, 64), _F32)
        aaug_o[...] = jnp.concatenate([_dot(xb, wa_r[...]), ones], axis=1)
        baug_o[...] = jnp.concatenate([_dot(xb, wb_r[...]), ones], axis=1)
        h1_o[...] = _dot(xb, wg_r[...])
        u_o[...] = _dot(xb, wu_r[...])
        v_o[...] = _dot(xb, wv_r[...])

    return pl.pallas_call(
        body,
        grid=(NB,),
        in_specs=[
            pl.BlockSpec((RB, 128), lambda i: (i, 0)),
            pl.BlockSpec((128, 64), lambda i: (0, 0)),
            pl.BlockSpec((128, 64), lambda i: (0, 0)),
            pl.BlockSpec((128, 64), lambda i: (0, 0)),
            pl.BlockSpec((128, 128), lambda i: (0, 0)),
            pl.BlockSpec((128, 128), lambda i: (0, 0)),
        ],
        out_specs=[
            pl.BlockSpec((RB, 128), lambda i: (i, 0)),
            pl.BlockSpec((RB, 128), lambda i: (i, 0)),
            pl.BlockSpec((RB, 64), lambda i: (i, 0)),
            pl.BlockSpec((RB, 128), lambda i: (i, 0)),
            pl.BlockSpec((RB, 128), lambda i: (i, 0)),
        ],
        out_shape=[_sds((N, 128)), _sds((N, 128)), _sds((N, 64)),
                   _sds((N, 128)), _sds((N, 128))],
    )(x, wa, wb, wg1, wu, wv)


def _tc_attn_combine(s1a, s1b, s2a, s2b, aaug, baug, h1, bea):
    def body(s1a_r, s1b_r, s2a_r, s2b_r, a_r, b_r, h1_r, bea_r, g1_o, agg_o):
        s1 = s1a_r[...] + s1b_r[...]
        s2 = s2a_r[...] + s2b_r[...]
        a = a_r[...][:, :64]
        b = b_r[...][:, :64]
        cs = s1[:, 64:65]
        co = s2[:, 64:65]
        bea_v = bea_r[...]
        mean_s = (cs * (a + bea_v) + s1[:, :64]) / jnp.maximum(cs, 1.0)
        mean_o = (s2[:, :64] + co * (b + bea_v)) / jnp.maximum(co, 1.0)
        agg_o[...] = _sigmoid(mean_s * mean_o)
        dis = lax.rsqrt(1.0 + co)
        g1_o[...] = jnp.concatenate(
            [h1_r[...] * dis, jnp.zeros((RB, 64), _F32)], axis=1)

    spec128 = pl.BlockSpec((RB, 128), lambda i: (i, 0))
    spec64 = pl.BlockSpec((RB, 64), lambda i: (i, 0))
    return pl.pallas_call(
        body,
        grid=(NB,),
        in_specs=[spec128, spec128, spec128, spec128, spec128, spec128, spec64,
                  pl.BlockSpec((1, 64), lambda i: (0, 0))],
        out_specs=[spec128, spec64],
        out_shape=[_sds((N, 128)), _sds((N, 64))],
    )(s1a, s1b, s2a, s2b, aaug, baug, h1, bea)


def _tc_gcn1(m1a, m1b, g1, agg, s2a, s2b, wg2, bg1):
    def body(m1a_r, m1b_r, g1_r, agg_r, s2a_r, s2b_r, wg2_r, bg1_r, g2_o):
        co = s2a_r[...][:, 64:65] + s2b_r[...][:, 64:65]
        dis = lax.rsqrt(1.0 + co)
        m1 = (m1a_r[...] + m1b_r[...])[:, :64]
        g1v = g1_r[...][:, :64]
        h = jnp.maximum(dis * (m1 + g1v) + bg1_r[...], 0.0) * agg_r[...]
        g2_o[...] = _dot(h, wg2_r[...]) * dis

    spec128 = pl.BlockSpec((RB, 128), lambda i: (i, 0))
    spec64 = pl.BlockSpec((RB, 64), lambda i: (i, 0))
    return pl.pallas_call(
        body,
        grid=(NB,),
        in_specs=[spec128, spec128, spec128, spec64, spec128, spec128,
                  pl.BlockSpec((64, 128), lambda i: (0, 0)),
                  pl.BlockSpec((1, 64), lambda i: (0, 0))],
        out_specs=pl.BlockSpec((RB, 128), lambda i: (i, 0)),
        out_shape=_sds((N, 128)),
    )(m1a, m1b, g1, agg, s2a, s2b, wg2, bg1)


def _tc_gcn2_heads(m2a, m2b, g2, s2a, s2b, u, v,
                   wna, bna, wpq, bnir_unused, wn1k, beta_n, wn2, bg2):
    def body(m2a_r, m2b_r, g2_r, s2a_r, s2b_r, u_r, v_r,
             wna_r, bna_r, wpq_r, wn1k_r, bn_r, wn2_r, bg2_r,
             upi_o, vqi_o, nl_o):
        co = s2a_r[...][:, 64:65] + s2b_r[...][:, 64:65]
        dis = lax.rsqrt(1.0 + co)
        h2 = jnp.maximum(dis * (m2a_r[...] + m2b_r[...] + g2_r[...]) + bg2_r[...], 0.0)
        ni = jnp.maximum(_dot(h2, wna_r[...]) + bna_r[...], 0.0)
        pq = _dot(ni, wpq_r[...])
        upi_o[...] = _pack2(u_r[...], pq[:, :128])
        vqi_o[...] = _pack2(v_r[...], pq[:, 128:])
        nx = _dot(h2, wn1k_r[...]) + bn_r[...]
        nx = jnp.where(nx > 0, nx, 0.2 * nx)
        logits = _dot(nx, wn2_r[...])
        logits = logits - jnp.max(logits, axis=1, keepdims=True)
        el = jnp.exp(logits)
        nl_o[...] = el / jnp.sum(el, axis=1, keepdims=True)

    spec128 = pl.BlockSpec((RB, 128), lambda i: (i, 0))
    return pl.pallas_call(
        body,
        grid=(NB,),
        in_specs=[spec128, spec128, spec128, spec128, spec128, spec128, spec128,
                  pl.BlockSpec((128, 128), lambda i: (0, 0)),
                  pl.BlockSpec((1, 128), lambda i: (0, 0)),
                  pl.BlockSpec((128, 256), lambda i: (0, 0)),
                  pl.BlockSpec((128, 64), lambda i: (0, 0)),
                  pl.BlockSpec((1, 64), lambda i: (0, 0)),
                  pl.BlockSpec((64, 160), lambda i: (0, 0)),
                  pl.BlockSpec((1, 128), lambda i: (0, 0))],
        out_specs=[spec128, spec128, pl.BlockSpec((RB, 160), lambda i: (i, 0))],
        out_shape=[_sds((N, 128), jnp.int32), _sds((N, 128), jnp.int32),
                   _sds((N, 160))],
    )(m2a, m2b, g2, s2a, s2b, u, v, wna, bna, wpq, wn1k, beta_n, wn2, bg2)


def _tc_edge_head(g1i, g2i, wm2, bm2, we1k, beta_e, we2p, bm1, bnir):
    def body(g1_r, g2_r, wm2_r, bm2_r, we1_r, be_r, we2_r, bm1_r, bnir_r,
             z_o, m_o, s_o):
        i = pl.program_id(0)
        w1 = g1_r[...]
        w2 = g2_r[...]
        t_pre = _unpack_hi(w1) + _unpack_hi(w2) + bm1_r[...]
        a_pre = _unpack_lo(w1) + _unpack_lo(w2) + bnir_r[...]
        t = jnp.maximum(t_pre, 0.0) * _sigmoid(a_pre)
        ef = jnp.maximum(_dot(t, wm2_r[...]) + bm2_r[...], 0.0)
        ex = _dot(ef, we1_r[...]) + be_r[...]
        ex = jnp.where(ex > 0, ex, 0.2 * ex)
        z = _dot(ex, we2_r[...])
        z_o[...] = z
        bmax = jnp.max(z, axis=0, keepdims=True)

        @pl.when(i == 0)
        def _():
            m_o[...] = jnp.broadcast_to(bmax, (8, 32))
            s_o[...] = jnp.broadcast_to(
                jnp.sum(jnp.exp(z - bmax), axis=0, keepdims=True), (8, 32))

        @pl.when(i > 0)
        def _():
            m_old = m_o[0:1, :]
            s_old = s_o[0:1, :]
            m_new = jnp.maximum(m_old, bmax)
            s_new = s_old * jnp.exp(m_old - m_new) + jnp.sum(
                jnp.exp(z - m_new), axis=0, keepdims=True)
            m_o[...] = jnp.broadcast_to(m_new, (8, 32))
            s_o[...] = jnp.broadcast_to(s_new, (8, 32))

    speci = pl.BlockSpec((EBLK, 128), lambda i: (i, 0))
    return pl.pallas_call(
        body,
        grid=(NEB,),
        in_specs=[speci, speci,
                  pl.BlockSpec((128, 256), lambda i: (0, 0)),
                  pl.BlockSpec((1, 256), lambda i: (0, 0)),
                  pl.BlockSpec((256, 128), lambda i: (0, 0)),
                  pl.BlockSpec((1, 128), lambda i: (0, 0)),
                  pl.BlockSpec((128, 32), lambda i: (0, 0)),
                  pl.BlockSpec((1, 128), lambda i: (0, 0)),
                  pl.BlockSpec((1, 128), lambda i: (0, 0))],
        out_specs=[pl.BlockSpec((EBLK, 32), lambda i: (i, 0)),
                   pl.BlockSpec((8, 32), lambda i: (0, 0)),
                   pl.BlockSpec((8, 32), lambda i: (0, 0))],
        out_shape=[_sds((E, 32)), _sds((8, 32)), _sds((8, 32))],
    )(g1i, g2i, wm2, bm2, we1k, beta_e, we2p, bm1, bnir)


def _tc_edge_softmax(z, m, s):
    def body(z_r, m_r, s_r, out_o):
        out_o[...] = jnp.exp(z_r[...] - m_r[0:1, :]) / s_r[0:1, :]

    return pl.pallas_call(
        body,
        grid=(NEB,),
        in_specs=[pl.BlockSpec((EBLK, 32), lambda i: (i, 0)),
                  pl.BlockSpec((8, 32), lambda i: (0, 0)),
                  pl.BlockSpec((8, 32), lambda i: (0, 0))],
        out_specs=pl.BlockSpec((EBLK, 32), lambda i: (i, 0)),
        out_shape=_sds((E, 32)),
    )(z, m, s)


def kernel(node_feats, edge_index, Wg1, bg1, Wg2, bg2, Wea, bea, Wna, bna,
           Wnir, bnir, Wm1, bm1, Wm2, bm2, Wn1, gamma_n, beta_n, Wn2,
           We1, gamma_e, beta_e, We2):
    x = node_feats
    subj = edge_index[:, 0]
    obj = edge_index[:, 1]

    wa = Wea[:128] - Wea[128:]
    wb = Wea[128:]
    wu = Wm1[:128] - Wm1[128:]
    wv = Wm1[128:]
    wpq = jnp.concatenate([Wnir[:128], Wnir[128:]], axis=1)
    kn = gamma_n / jnp.sqrt(1.0 + 1e-5)
    wn1k = Wn1 * kn[None, :]
    ke = gamma_e / jnp.sqrt(1.0 + 1e-5)
    we1k = We1 * ke[None, :]
    we2p = jnp.concatenate([We2, jnp.zeros((128, 5), _F32)], axis=1)
    r = lambda v: v.reshape(1, -1)
    zeros128 = jnp.zeros((SLAB, 128), _F32)

    aaug, baug, h1, u, v = _tc_proj(x, wa, wb, Wg1, wu, wv)
    s1p = _sc_segment_sum(baug, obj, subj, zeros128, 128)
    s2p = _sc_segment_sum(aaug, subj, obj, zeros128, 128)
    g1, agg = _tc_attn_combine(s1p[0], s1p[1], s2p[0], s2p[1],
                               aaug, baug, h1, r(bea))
    m1p = _sc_segment_sum(g1, subj, obj, zeros128, 128)
    g2 = _tc_gcn1(m1p[0], m1p[1], g1, agg, s2p[0], s2p[1], Wg2, r(bg1))
    m2p = _sc_segment_sum(g2, subj, obj, zeros128, 128)
    upi, vqi, node_logits = _tc_gcn2_heads(
        m2p[0], m2p[1], g2, s2p[0], s2p[1], u, v,
        Wna, r(bna), wpq, None, wn1k, r(beta_n), Wn2, r(bg2))
    g1i, g2i = _sc_edge_gather(upi, vqi, subj, obj)
    z, m, s = _tc_edge_head(g1i, g2i, Wm2, r(bm2), we1k, r(beta_e),
                            we2p, r(bm1), r(bnir))
    e32 = _tc_edge_softmax(z, m, s)
    edge_logits = e32[:, :27]
    return node_logits, edge_logits


# speedup vs baseline: 5.5320x; 1.0855x over previous
import functools

import jax
import jax.numpy as jnp
from jax import lax
from jax.experimental import pallas as pl
from jax.experimental.pallas import tpu as pltpu
from jax.experimental.pallas import tpu_sc as plsc

N = 10000
E = 320000
NC = 2
NS = 16
NW = NC * NS
EW = E // NW
CH = 80
NCHUNK = EW // CH
SLOTS = 5
NGRP = NCHUNK // SLOTS
CHS = 40
NGRPS = EW // CHS // SLOTS
NPAD = 10240
SLAB = NPAD // NS

RB = 400
NB = N // RB
EBLK = 512
NEB = E // EBLK

_F32 = jnp.float32
_MASK_HI = -65536


def _sds(shape, dtype=_F32):
    return jax.ShapeDtypeStruct(shape, dtype)


def _sigmoid(x):
    return 1.0 / (1.0 + jnp.exp(-x))


def _bf16_bits(x):
    b = lax.bitcast_convert_type(x, jnp.int32)
    return b + 0x7FFF + (lax.shift_right_logical(b, 16) & 1)


def _pack2(hi, lo):
    return (_bf16_bits(hi) & _MASK_HI) | lax.shift_right_logical(_bf16_bits(lo), 16)


def _unpack_hi(w):
    return lax.bitcast_convert_type(w & _MASK_HI, _F32)


def _unpack_lo(w):
    return lax.bitcast_convert_type(jnp.left_shift(w, 16), _F32)


def _mesh():
    return plsc.VectorSubcoreMesh(
        core_axis_name="c", subcore_axis_name="s", num_cores=NC, num_subcores=NS)


def _sc_segment_sum(table, gather_idx, scatter_idx, zeros):
    @functools.partial(
        pl.kernel,
        out_type=_sds((NC, NPAD, 128)),
        mesh=_mesh(),
        scratch_types=[
            pltpu.VMEM((SLOTS, CHS), jnp.int32),
            pltpu.VMEM((SLOTS, CHS), jnp.int32),
            pltpu.VMEM((SLOTS, CHS, 128), _F32),
            pltpu.VMEM_SHARED((NPAD, 128), _F32),
        ] + [pltpu.SemaphoreType.DMA] * SLOTS,
    )
    def body(tab_hbm, g_hbm, s_hbm, z_hbm, out_hbm, gidx, sidx, rows, acc, *sems):
        cid = lax.axis_index("c")
        sid = lax.axis_index("s")
        wid = cid * NS + sid
        pltpu.sync_copy(z_hbm, acc.at[pl.ds(sid * SLAB, SLAB), :])
        plsc.subcore_barrier()
        base = wid * EW

        def group(g, carry):
            c0 = g * SLOTS
            descs = []
            for k in range(SLOTS):
                off = base + (c0 + k) * CHS
                pltpu.sync_copy(g_hbm.at[pl.ds(off, CHS)], gidx.at[k])
                pltpu.sync_copy(s_hbm.at[pl.ds(off, CHS)], sidx.at[k])
                descs.append(
                    pltpu.async_copy(tab_hbm.at[gidx.at[k]], rows.at[k], sems[k]))
            for k in range(SLOTS):
                descs[k].wait()
                pltpu.sync_copy(rows.at[k], acc.at[sidx.at[k]], add=True)
            return carry

        lax.fori_loop(0, NGRPS, group, 0)
        plsc.subcore_barrier()
        sl = pl.ds(sid * SLAB, SLAB)
        pltpu.sync_copy(acc.at[sl, :], out_hbm.at[cid, sl, :])

    return body(table, gather_idx, scatter_idx, zeros)


def _sc_edge_gather(upi, vqi, subj, obj):
    @functools.partial(
        pl.kernel,
        out_type=[_sds((E, 128), jnp.int32), _sds((E, 128), jnp.int32)],
        mesh=_mesh(),
        scratch_types=[
            pltpu.VMEM((SLOTS, CH), jnp.int32),
            pltpu.VMEM((SLOTS, CH), jnp.int32),
            pltpu.VMEM((SLOTS, CH, 128), jnp.int32),
            pltpu.VMEM((SLOTS, CH, 128), jnp.int32),
        ] + [pltpu.SemaphoreType.DMA] * (2 * SLOTS),
    )
    def body(up_hbm, vq_hbm, s_hbm, o_hbm, g1_hbm, g2_hbm,
             sidx, oidx, r1, r2, *sems):
        cid = lax.axis_index("c")
        sid = lax.axis_index("s")
        wid = cid * NS + sid
        base = wid * EW

        def group(g, carry):
            c0 = g * SLOTS
            descs = []
            for k in range(SLOTS):
                off = base + (c0 + k) * CH
                pltpu.sync_copy(s_hbm.at[pl.ds(off, CH)], sidx.at[k])
                pltpu.sync_copy(o_hbm.at[pl.ds(off, CH)], oidx.at[k])
                descs.append(
                    pltpu.async_copy(up_hbm.at[sidx.at[k]], r1.at[k], sems[2 * k]))
                descs.append(
                    pltpu.async_copy(vq_hbm.at[oidx.at[k]], r2.at[k], sems[2 * k + 1]))
            for k in range(SLOTS):
                off = base + (c0 + k) * CH
                descs[2 * k].wait()
                pltpu.sync_copy(r1.at[k], g1_hbm.at[pl.ds(off, CH), :])
                descs[2 * k + 1].wait()
                pltpu.sync_copy(r2.at[k], g2_hbm.at[pl.ds(off, CH), :])
            return carry

        lax.fori_loop(0, NGRP, group, 0)

    return body(upi, vqi, subj, obj)


def _dot(a, b):
    return jnp.dot(a, b, preferred_element_type=_F32)


def _tc_proj(x, wa, wb, wg1, wu, wv):
    def body(x_r, wa_r, wb_r, wg_r, wu_r, wv_r, aaug_o, baug_o, h1_o, u_o, v_o):
        xb = x_r[...]
        ones = jnp.ones((RB, 64), _F32)
        aaug_o[...] = jnp.concatenate([_dot(xb, wa_r[...]), ones], axis=1)
        baug_o[...] = jnp.concatenate([_dot(xb, wb_r[...]), ones], axis=1)
        h1_o[...] = _dot(xb, wg_r[...])
        u_o[...] = _dot(xb, wu_r[...])
        v_o[...] = _dot(xb, wv_r[...])

    return pl.pallas_call(
        body,
        grid=(NB,),
        in_specs=[
            pl.BlockSpec((RB, 128), lambda i: (i, 0)),
            pl.BlockSpec((128, 64), lambda i: (0, 0)),
            pl.BlockSpec((128, 64), lambda i: (0, 0)),
            pl.BlockSpec((128, 64), lambda i: (0, 0)),
            pl.BlockSpec((128, 128), lambda i: (0, 0)),
            pl.BlockSpec((128, 128), lambda i: (0, 0)),
        ],
        out_specs=[
            pl.BlockSpec((RB, 128), lambda i: (i, 0)),
            pl.BlockSpec((RB, 128), lambda i: (i, 0)),
            pl.BlockSpec((RB, 64), lambda i: (i, 0)),
            pl.BlockSpec((RB, 128), lambda i: (i, 0)),
            pl.BlockSpec((RB, 128), lambda i: (i, 0)),
        ],
        out_shape=[_sds((N, 128)), _sds((N, 128)), _sds((N, 64)),
                   _sds((N, 128)), _sds((N, 128))],
    )(x, wa, wb, wg1, wu, wv)


def _tc_attn_combine(s1a, s1b, s2a, s2b, aaug, baug, h1, bea):
    def body(s1a_r, s1b_r, s2a_r, s2b_r, a_r, b_r, h1_r, bea_r, g1_o, agg_o):
        s1 = s1a_r[...] + s1b_r[...]
        s2 = s2a_r[...] + s2b_r[...]
        a = a_r[...][:, :64]
        b = b_r[...][:, :64]
        cs = s1[:, 64:65]
        co = s2[:, 64:65]
        bea_v = bea_r[...]
        mean_s = (cs * (a + bea_v) + s1[:, :64]) / jnp.maximum(cs, 1.0)
        mean_o = (s2[:, :64] + co * (b + bea_v)) / jnp.maximum(co, 1.0)
        agg_o[...] = _sigmoid(mean_s * mean_o)
        dis = lax.rsqrt(1.0 + co)
        g1_o[...] = jnp.concatenate(
            [h1_r[...] * dis, jnp.zeros((RB, 64), _F32)], axis=1)

    spec128 = pl.BlockSpec((RB, 128), lambda i: (i, 0))
    spec64 = pl.BlockSpec((RB, 64), lambda i: (i, 0))
    return pl.pallas_call(
        body,
        grid=(NB,),
        in_specs=[spec128, spec128, spec128, spec128, spec128, spec128, spec64,
                  pl.BlockSpec((1, 64), lambda i: (0, 0))],
        out_specs=[spec128, spec64],
        out_shape=[_sds((N, 128)), _sds((N, 64))],
    )(s1a, s1b, s2a, s2b, aaug, baug, h1, bea)


def _tc_gcn1(m1a, m1b, g1, agg, s2a, s2b, wg2, bg1):
    def body(m1a_r, m1b_r, g1_r, agg_r, s2a_r, s2b_r, wg2_r, bg1_r, g2_o):
        co = s2a_r[...][:, 64:65] + s2b_r[...][:, 64:65]
        dis = lax.rsqrt(1.0 + co)
        m1 = (m1a_r[...] + m1b_r[...])[:, :64]
        g1v = g1_r[...][:, :64]
        h = jnp.maximum(dis * (m1 + g1v) + bg1_r[...], 0.0) * agg_r[...]
        g2_o[...] = _dot(h, wg2_r[...]) * dis

    spec128 = pl.BlockSpec((RB, 128), lambda i: (i, 0))
    spec64 = pl.BlockSpec((RB, 64), lambda i: (i, 0))
    return pl.pallas_call(
        body,
        grid=(NB,),
        in_specs=[spec128, spec128, spec128, spec64, spec128, spec128,
                  pl.BlockSpec((64, 128), lambda i: (0, 0)),
                  pl.BlockSpec((1, 64), lambda i: (0, 0))],
        out_specs=pl.BlockSpec((RB, 128), lambda i: (i, 0)),
        out_shape=_sds((N, 128)),
    )(m1a, m1b, g1, agg, s2a, s2b, wg2, bg1)


def _tc_gcn2_heads(m2a, m2b, g2, s2a, s2b, u, v,
                   wna, bna, wpq, bnir_unused, wn1k, beta_n, wn2, bg2):
    def body(m2a_r, m2b_r, g2_r, s2a_r, s2b_r, u_r, v_r,
             wna_r, bna_r, wpq_r, wn1k_r, bn_r, wn2_r, bg2_r,
             upi_o, vqi_o, nl_o):
        co = s2a_r[...][:, 64:65] + s2b_r[...][:, 64:65]
        dis = lax.rsqrt(1.0 + co)
        h2 = jnp.maximum(dis * (m2a_r[...] + m2b_r[...] + g2_r[...]) + bg2_r[...], 0.0)
        ni = jnp.maximum(_dot(h2, wna_r[...]) + bna_r[...], 0.0)
        pq = _dot(ni, wpq_r[...])
        upi_o[...] = _pack2(u_r[...], pq[:, :128])
        vqi_o[...] = _pack2(v_r[...], pq[:, 128:])
        nx = _dot(h2, wn1k_r[...]) + bn_r[...]
        nx = jnp.where(nx > 0, nx, 0.2 * nx)
        logits = _dot(nx, wn2_r[...])
        logits = logits - jnp.max(logits, axis=1, keepdims=True)
        el = jnp.exp(logits)
        nl_o[...] = el / jnp.sum(el, axis=1, keepdims=True)

    spec128 = pl.BlockSpec((RB, 128), lambda i: (i, 0))
    return pl.pallas_call(
        body,
        grid=(NB,),
        in_specs=[spec128, spec128, spec128, spec128, spec128, spec128, spec128,
                  pl.BlockSpec((128, 128), lambda i: (0, 0)),
                  pl.BlockSpec((1, 128), lambda i: (0, 0)),
                  pl.BlockSpec((128, 256), lambda i: (0, 0)),
                  pl.BlockSpec((128, 64), lambda i: (0, 0)),
                  pl.BlockSpec((1, 64), lambda i: (0, 0)),
                  pl.BlockSpec((64, 160), lambda i: (0, 0)),
                  pl.BlockSpec((1, 128), lambda i: (0, 0))],
        out_specs=[spec128, spec128, pl.BlockSpec((RB, 160), lambda i: (i, 0))],
        out_shape=[_sds((N, 128), jnp.int32), _sds((N, 128), jnp.int32),
                   _sds((N, 160))],
    )(m2a, m2b, g2, s2a, s2b, u, v, wna, bna, wpq, wn1k, beta_n, wn2, bg2)


def _tc_edge_head(g1i, g2i, wm2, bm2, we1k, beta_e, we2p, bm1, bnir):
    def body(g1_r, g2_r, wm2_r, bm2_r, we1_r, be_r, we2_r, bm1_r, bnir_r,
             z_o, m_o, s_o):
        i = pl.program_id(0)
        w1 = g1_r[...]
        w2 = g2_r[...]
        t_pre = _unpack_hi(w1) + _unpack_hi(w2) + bm1_r[...]
        a_pre = _unpack_lo(w1) + _unpack_lo(w2) + bnir_r[...]
        t = jnp.maximum(t_pre, 0.0) * _sigmoid(a_pre)
        ef = jnp.maximum(_dot(t, wm2_r[...]) + bm2_r[...], 0.0)
        ex = _dot(ef, we1_r[...]) + be_r[...]
        ex = jnp.where(ex > 0, ex, 0.2 * ex)
        z = _dot(ex, we2_r[...])
        z_o[...] = z
        bmax = jnp.max(z, axis=0, keepdims=True)

        @pl.when(i == 0)
        def _():
            m_o[...] = jnp.broadcast_to(bmax, (8, 32))
            s_o[...] = jnp.broadcast_to(
                jnp.sum(jnp.exp(z - bmax), axis=0, keepdims=True), (8, 32))

        @pl.when(i > 0)
        def _():
            m_old = m_o[0:1, :]
            s_old = s_o[0:1, :]
            m_new = jnp.maximum(m_old, bmax)
            s_new = s_old * jnp.exp(m_old - m_new) + jnp.sum(
                jnp.exp(z - m_new), axis=0, keepdims=True)
            m_o[...] = jnp.broadcast_to(m_new, (8, 32))
            s_o[...] = jnp.broadcast_to(s_new, (8, 32))

    speci = pl.BlockSpec((EBLK, 128), lambda i: (i, 0))
    return pl.pallas_call(
        body,
        grid=(NEB,),
        in_specs=[speci, speci,
                  pl.BlockSpec((128, 256), lambda i: (0, 0)),
                  pl.BlockSpec((1, 256), lambda i: (0, 0)),
                  pl.BlockSpec((256, 128), lambda i: (0, 0)),
                  pl.BlockSpec((1, 128), lambda i: (0, 0)),
                  pl.BlockSpec((128, 32), lambda i: (0, 0)),
                  pl.BlockSpec((1, 128), lambda i: (0, 0)),
                  pl.BlockSpec((1, 128), lambda i: (0, 0))],
        out_specs=[pl.BlockSpec((EBLK, 32), lambda i: (i, 0)),
                   pl.BlockSpec((8, 32), lambda i: (0, 0)),
                   pl.BlockSpec((8, 32), lambda i: (0, 0))],
        out_shape=[_sds((E, 32)), _sds((8, 32)), _sds((8, 32))],
    )(g1i, g2i, wm2, bm2, we1k, beta_e, we2p, bm1, bnir)


def _tc_edge_softmax(z, m, s):
    def body(z_r, m_r, s_r, out_o):
        out_o[...] = jnp.exp(z_r[...] - m_r[0:1, :]) / s_r[0:1, :]

    return pl.pallas_call(
        body,
        grid=(NEB,),
        in_specs=[pl.BlockSpec((EBLK, 32), lambda i: (i, 0)),
                  pl.BlockSpec((8, 32), lambda i: (0, 0)),
                  pl.BlockSpec((8, 32), lambda i: (0, 0))],
        out_specs=pl.BlockSpec((EBLK, 32), lambda i: (i, 0)),
        out_shape=_sds((E, 32)),
    )(z, m, s)


def kernel(node_feats, edge_index, Wg1, bg1, Wg2, bg2, Wea, bea, Wna, bna,
           Wnir, bnir, Wm1, bm1, Wm2, bm2, Wn1, gamma_n, beta_n, Wn2,
           We1, gamma_e, beta_e, We2):
    x = node_feats
    subj = edge_index[:, 0]
    obj = edge_index[:, 1]

    wa = Wea[:128] - Wea[128:]
    wb = Wea[128:]
    wu = Wm1[:128] - Wm1[128:]
    wv = Wm1[128:]
    wpq = jnp.concatenate([Wnir[:128], Wnir[128:]], axis=1)
    kn = gamma_n / jnp.sqrt(1.0 + 1e-5)
    wn1k = Wn1 * kn[None, :]
    ke = gamma_e / jnp.sqrt(1.0 + 1e-5)
    we1k = We1 * ke[None, :]
    we2p = jnp.concatenate([We2, jnp.zeros((128, 5), _F32)], axis=1)
    r = lambda v: v.reshape(1, -1)
    zeros128 = jnp.zeros((SLAB, 128), _F32)

    aaug, baug, h1, u, v = _tc_proj(x, wa, wb, Wg1, wu, wv)
    s1p = _sc_segment_sum(baug, obj, subj, zeros128)
    s2p = _sc_segment_sum(aaug, subj, obj, zeros128)
    g1, agg = _tc_attn_combine(s1p[0], s1p[1], s2p[0], s2p[1],
                               aaug, baug, h1, r(bea))
    m1p = _sc_segment_sum(g1, subj, obj, zeros128)
    g2 = _tc_gcn1(m1p[0], m1p[1], g1, agg, s2p[0], s2p[1], Wg2, r(bg1))
    m2p = _sc_segment_sum(g2, subj, obj, zeros128)
    upi, vqi, node_logits = _tc_gcn2_heads(
        m2p[0], m2p[1], g2, s2p[0], s2p[1], u, v,
        Wna, r(bna), wpq, None, wn1k, r(beta_n), Wn2, r(bg2))
    g1i, g2i = _sc_edge_gather(upi, vqi, subj, obj)
    z, m, s = _tc_edge_head(g1i, g2i, Wm2, r(bm2), we1k, r(beta_e),
                            we2p, r(bm1), r(bnir))
    e32 = _tc_edge_softmax(z, m, s)
    edge_logits = e32[:, :27]
    return node_logits, edge_logits


# speedup vs baseline: 6.7495x; 1.2201x over previous
import functools

import jax
import jax.numpy as jnp
from jax import lax
from jax.experimental import pallas as pl
from jax.experimental.pallas import tpu as pltpu
from jax.experimental.pallas import tpu_sc as plsc

N = 10000
E = 320000
NC = 2
NS = 16
NW = NC * NS
EW = E // NW
CH = 80
NCHUNK = EW // CH
SLOTS = 5
NGRP = NCHUNK // SLOTS
CHS = 40
NCHS = EW // CHS
NGRPS = NCHS // SLOTS
NPAD = 10240
SLAB = NPAD // NS

RB = 400
NB = N // RB
EBLK = 512
NEB = E // EBLK

_F32 = jnp.float32
_MASK_HI = -65536


def _sds(shape, dtype=_F32):
    return jax.ShapeDtypeStruct(shape, dtype)


def _sigmoid(x):
    return 1.0 / (1.0 + jnp.exp(-x))


def _bf16_bits(x):
    b = lax.bitcast_convert_type(x, jnp.int32)
    return b + 0x7FFF + (lax.shift_right_logical(b, 16) & 1)


def _pack2(hi, lo):
    return (_bf16_bits(hi) & _MASK_HI) | lax.shift_right_logical(_bf16_bits(lo), 16)


def _unpack_hi(w):
    return lax.bitcast_convert_type(w & _MASK_HI, _F32)


def _unpack_lo(w):
    return lax.bitcast_convert_type(jnp.left_shift(w, 16), _F32)


def _mesh():
    return plsc.VectorSubcoreMesh(
        core_axis_name="c", subcore_axis_name="s", num_cores=NC, num_subcores=NS)


def _sc_segment_sum(table, gather_idx, scatter_idx, zeros):
    @functools.partial(
        pl.kernel,
        out_type=_sds((NC, NPAD, 128)),
        mesh=_mesh(),
        scratch_types=[
            pltpu.VMEM((EW,), jnp.int32),
            pltpu.VMEM((SLOTS, CHS), jnp.int32),
            pltpu.VMEM((SLOTS, CHS, 128), _F32),
            pltpu.VMEM_SHARED((NPAD, 128), _F32),
        ] + [pltpu.SemaphoreType.DMA] * (2 * SLOTS),
    )
    def body(tab_hbm, g_hbm, s_hbm, z_hbm, out_hbm, gflat, sidx, rows, acc, *sems):
        gsem = sems[:SLOTS]
        ssem = sems[SLOTS:]
        cid = lax.axis_index("c")
        sid = lax.axis_index("s")
        wid = cid * NS + sid
        base = wid * EW
        pltpu.sync_copy(z_hbm, acc.at[pl.ds(sid * SLAB, SLAB), :])
        pltpu.sync_copy(g_hbm.at[pl.ds(base, EW)], gflat)
        plsc.subcore_barrier()

        def group(g, carry):
            c0 = g * SLOTS
            descs = []
            for k in range(SLOTS):
                @pl.when(g > 0)
                def _(k=k):
                    pltpu.make_async_copy(
                        rows.at[k], acc.at[sidx.at[k]], ssem[k]).wait()
                pltpu.sync_copy(
                    s_hbm.at[pl.ds(base + (c0 + k) * CHS, CHS)], sidx.at[k])
                descs.append(
                    pltpu.async_copy(
                        tab_hbm.at[gflat.at[pl.ds((c0 + k) * CHS, CHS)]],
                        rows.at[k], gsem[k]))
            for k in range(SLOTS):
                descs[k].wait()
                pltpu.async_copy(rows.at[k], acc.at[sidx.at[k]], ssem[k],
                                 add=True)
            return carry

        lax.fori_loop(0, NGRPS, group, 0)
        for k in range(SLOTS):
            pltpu.make_async_copy(rows.at[k], acc.at[sidx.at[k]], ssem[k]).wait()
        plsc.subcore_barrier()
        sl = pl.ds(sid * SLAB, SLAB)
        pltpu.sync_copy(acc.at[sl, :], out_hbm.at[cid, sl, :])

    return body(table, gather_idx, scatter_idx, zeros)


def _sc_edge_gather(upi, vqi, subj, obj):
    @functools.partial(
        pl.kernel,
        out_type=[_sds((E, 128), jnp.int32), _sds((E, 128), jnp.int32)],
        mesh=_mesh(),
        scratch_types=[
            pltpu.VMEM((EW,), jnp.int32),
            pltpu.VMEM((EW,), jnp.int32),
            pltpu.VMEM((SLOTS, CH, 128), jnp.int32),
            pltpu.VMEM((SLOTS, CH, 128), jnp.int32),
        ] + [pltpu.SemaphoreType.DMA] * (4 * SLOTS),
    )
    def body(up_hbm, vq_hbm, s_hbm, o_hbm, g1_hbm, g2_hbm,
             sflat, oflat, r1, r2, *sems):
        gs1 = sems[:SLOTS]
        gs2 = sems[SLOTS:2 * SLOTS]
        ws1 = sems[2 * SLOTS:3 * SLOTS]
        ws2 = sems[3 * SLOTS:]
        cid = lax.axis_index("c")
        sid = lax.axis_index("s")
        wid = cid * NS + sid
        base = wid * EW
        pltpu.sync_copy(s_hbm.at[pl.ds(base, EW)], sflat)
        pltpu.sync_copy(o_hbm.at[pl.ds(base, EW)], oflat)

        def group(g, carry):
            c0 = g * SLOTS
            descs = []
            for k in range(SLOTS):
                @pl.when(g > 0)
                def _(k=k, c0=c0):
                    offp = base + (c0 - SLOTS + k) * CH
                    pltpu.make_async_copy(
                        r1.at[k], g1_hbm.at[pl.ds(offp, CH), :], ws1[k]).wait()
                    pltpu.make_async_copy(
                        r2.at[k], g2_hbm.at[pl.ds(offp, CH), :], ws2[k]).wait()
                sl = pl.ds((c0 + k) * CH, CH)
                descs.append(
                    pltpu.async_copy(up_hbm.at[sflat.at[sl]], r1.at[k], gs1[k]))
                descs.append(
                    pltpu.async_copy(vq_hbm.at[oflat.at[sl]], r2.at[k], gs2[k]))
            for k in range(SLOTS):
                off = base + (c0 + k) * CH
                descs[2 * k].wait()
                pltpu.async_copy(r1.at[k], g1_hbm.at[pl.ds(off, CH), :], ws1[k])
                descs[2 * k + 1].wait()
                pltpu.async_copy(r2.at[k], g2_hbm.at[pl.ds(off, CH), :], ws2[k])
            return carry

        lax.fori_loop(0, NGRP, group, 0)
        for k in range(SLOTS):
            offp = base + ((NGRP - 1) * SLOTS + k) * CH
            pltpu.make_async_copy(
                r1.at[k], g1_hbm.at[pl.ds(offp, CH), :], ws1[k]).wait()
            pltpu.make_async_copy(
                r2.at[k], g2_hbm.at[pl.ds(offp, CH), :], ws2[k]).wait()

    return body(upi, vqi, subj, obj)


def _dot(a, b):
    return jnp.dot(a, b, preferred_element_type=_F32)


def _tc_proj(x, wa, wb, wg1, wu, wv):
    def body(x_r, wa_r, wb_r, wg_r, wu_r, wv_r, aaug_o, baug_o, h1_o, u_o, v_o):
        xb = x_r[...]
        ones = jnp.ones((RB, 64), _F32)
        aaug_o[...] = jnp.concatenate([_dot(xb, wa_r[...]), ones], axis=1)
        baug_o[...] = jnp.concatenate([_dot(xb, wb_r[...]), ones], axis=1)
        h1_o[...] = _dot(xb, wg_r[...])
        u_o[...] = _dot(xb, wu_r[...])
        v_o[...] = _dot(xb, wv_r[...])

    return pl.pallas_call(
        body,
        grid=(NB,),
        in_specs=[
            pl.BlockSpec((RB, 128), lambda i: (i, 0)),
            pl.BlockSpec((128, 64), lambda i: (0, 0)),
            pl.BlockSpec((128, 64), lambda i: (0, 0)),
            pl.BlockSpec((128, 64), lambda i: (0, 0)),
            pl.BlockSpec((128, 128), lambda i: (0, 0)),
            pl.BlockSpec((128, 128), lambda i: (0, 0)),
        ],
        out_specs=[
            pl.BlockSpec((RB, 128), lambda i: (i, 0)),
            pl.BlockSpec((RB, 128), lambda i: (i, 0)),
            pl.BlockSpec((RB, 64), lambda i: (i, 0)),
            pl.BlockSpec((RB, 128), lambda i: (i, 0)),
            pl.BlockSpec((RB, 128), lambda i: (i, 0)),
        ],
        out_shape=[_sds((N, 128)), _sds((N, 128)), _sds((N, 64)),
                   _sds((N, 128)), _sds((N, 128))],
    )(x, wa, wb, wg1, wu, wv)


def _tc_attn_combine(s1a, s1b, s2a, s2b, aaug, baug, h1, bea):
    def body(s1a_r, s1b_r, s2a_r, s2b_r, a_r, b_r, h1_r, bea_r, g1_o, agg_o):
        s1 = s1a_r[...] + s1b_r[...]
        s2 = s2a_r[...] + s2b_r[...]
        a = a_r[...][:, :64]
        b = b_r[...][:, :64]
        cs = s1[:, 64:65]
        co = s2[:, 64:65]
        bea_v = bea_r[...]
        mean_s = (cs * (a + bea_v) + s1[:, :64]) / jnp.maximum(cs, 1.0)
        mean_o = (s2[:, :64] + co * (b + bea_v)) / jnp.maximum(co, 1.0)
        agg_o[...] = _sigmoid(mean_s * mean_o)
        dis = lax.rsqrt(1.0 + co)
        g1_o[...] = jnp.concatenate(
            [h1_r[...] * dis, jnp.zeros((RB, 64), _F32)], axis=1)

    spec128 = pl.BlockSpec((RB, 128), lambda i: (i, 0))
    spec64 = pl.BlockSpec((RB, 64), lambda i: (i, 0))
    return pl.pallas_call(
        body,
        grid=(NB,),
        in_specs=[spec128, spec128, spec128, spec128, spec128, spec128, spec64,
                  pl.BlockSpec((1, 64), lambda i: (0, 0))],
        out_specs=[spec128, spec64],
        out_shape=[_sds((N, 128)), _sds((N, 64))],
    )(s1a, s1b, s2a, s2b, aaug, baug, h1, bea)


def _tc_gcn1(m1a, m1b, g1, agg, s2a, s2b, wg2, bg1):
    def body(m1a_r, m1b_r, g1_r, agg_r, s2a_r, s2b_r, wg2_r, bg1_r, g2_o):
        co = s2a_r[...][:, 64:65] + s2b_r[...][:, 64:65]
        dis = lax.rsqrt(1.0 + co)
        m1 = (m1a_r[...] + m1b_r[...])[:, :64]
        g1v = g1_r[...][:, :64]
        h = jnp.maximum(dis * (m1 + g1v) + bg1_r[...], 0.0) * agg_r[...]
        g2_o[...] = _dot(h, wg2_r[...]) * dis

    spec128 = pl.BlockSpec((RB, 128), lambda i: (i, 0))
    spec64 = pl.BlockSpec((RB, 64), lambda i: (i, 0))
    return pl.pallas_call(
        body,
        grid=(NB,),
        in_specs=[spec128, spec128, spec128, spec64, spec128, spec128,
                  pl.BlockSpec((64, 128), lambda i: (0, 0)),
                  pl.BlockSpec((1, 64), lambda i: (0, 0))],
        out_specs=pl.BlockSpec((RB, 128), lambda i: (i, 0)),
        out_shape=_sds((N, 128)),
    )(m1a, m1b, g1, agg, s2a, s2b, wg2, bg1)


def _tc_gcn2_heads(m2a, m2b, g2, s2a, s2b, u, v,
                   wna, bna, wpq, bnir_unused, wn1k, beta_n, wn2, bg2):
    def body(m2a_r, m2b_r, g2_r, s2a_r, s2b_r, u_r, v_r,
             wna_r, bna_r, wpq_r, wn1k_r, bn_r, wn2_r, bg2_r,
             upi_o, vqi_o, nl_o):
        co = s2a_r[...][:, 64:65] + s2b_r[...][:, 64:65]
        dis = lax.rsqrt(1.0 + co)
        h2 = jnp.maximum(dis * (m2a_r[...] + m2b_r[...] + g2_r[...]) + bg2_r[...], 0.0)
        ni = jnp.maximum(_dot(h2, wna_r[...]) + bna_r[...], 0.0)
        pq = _dot(ni, wpq_r[...])
        upi_o[...] = _pack2(u_r[...], pq[:, :128])
        vqi_o[...] = _pack2(v_r[...], pq[:, 128:])
        nx = _dot(h2, wn1k_r[...]) + bn_r[...]
        nx = jnp.where(nx > 0, nx, 0.2 * nx)
        logits = _dot(nx, wn2_r[...])
        logits = logits - jnp.max(logits, axis=1, keepdims=True)
        el = jnp.exp(logits)
        nl_o[...] = el / jnp.sum(el, axis=1, keepdims=True)

    spec128 = pl.BlockSpec((RB, 128), lambda i: (i, 0))
    return pl.pallas_call(
        body,
        grid=(NB,),
        in_specs=[spec128, spec128, spec128, spec128, spec128, spec128, spec128,
                  pl.BlockSpec((128, 128), lambda i: (0, 0)),
                  pl.BlockSpec((1, 128), lambda i: (0, 0)),
                  pl.BlockSpec((128, 256), lambda i: (0, 0)),
                  pl.BlockSpec((128, 64), lambda i: (0, 0)),
                  pl.BlockSpec((1, 64), lambda i: (0, 0)),
                  pl.BlockSpec((64, 160), lambda i: (0, 0)),
                  pl.BlockSpec((1, 128), lambda i: (0, 0))],
        out_specs=[spec128, spec128, pl.BlockSpec((RB, 160), lambda i: (i, 0))],
        out_shape=[_sds((N, 128), jnp.int32), _sds((N, 128), jnp.int32),
                   _sds((N, 160))],
    )(m2a, m2b, g2, s2a, s2b, u, v, wna, bna, wpq, wn1k, beta_n, wn2, bg2)


def _tc_edge_head(g1i, g2i, wm2, bm2, we1k, beta_e, we2p, bm1, bnir):
    def body(g1_r, g2_r, wm2_r, bm2_r, we1_r, be_r, we2_r, bm1_r, bnir_r,
             z_o, m_o, s_o):
        i = pl.program_id(0)
        w1 = g1_r[...]
        w2 = g2_r[...]
        t_pre = _unpack_hi(w1) + _unpack_hi(w2) + bm1_r[...]
        a_pre = _unpack_lo(w1) + _unpack_lo(w2) + bnir_r[...]
        t = jnp.maximum(t_pre, 0.0) * _sigmoid(a_pre)
        ef = jnp.maximum(_dot(t, wm2_r[...]) + bm2_r[...], 0.0)
        ex = _dot(ef, we1_r[...]) + be_r[...]
        ex = jnp.where(ex > 0, ex, 0.2 * ex)
        z = _dot(ex, we2_r[...])
        z_o[...] = z
        bmax = jnp.max(z, axis=0, keepdims=True)

        @pl.when(i == 0)
        def _():
            m_o[...] = jnp.broadcast_to(bmax, (8, 32))
            s_o[...] = jnp.broadcast_to(
                jnp.sum(jnp.exp(z - bmax), axis=0, keepdims=True), (8, 32))

        @pl.when(i > 0)
        def _():
            m_old = m_o[0:1, :]
            s_old = s_o[0:1, :]
            m_new = jnp.maximum(m_old, bmax)
            s_new = s_old * jnp.exp(m_old - m_new) + jnp.sum(
                jnp.exp(z - m_new), axis=0, keepdims=True)
            m_o[...] = jnp.broadcast_to(m_new, (8, 32))
            s_o[...] = jnp.broadcast_to(s_new, (8, 32))

    speci = pl.BlockSpec((EBLK, 128), lambda i: (i, 0))
    return pl.pallas_call(
        body,
        grid=(NEB,),
        in_specs=[speci, speci,
                  pl.BlockSpec((128, 256), lambda i: (0, 0)),
                  pl.BlockSpec((1, 256), lambda i: (0, 0)),
                  pl.BlockSpec((256, 128), lambda i: (0, 0)),
                  pl.BlockSpec((1, 128), lambda i: (0, 0)),
                  pl.BlockSpec((128, 32), lambda i: (0, 0)),
                  pl.BlockSpec((1, 128), lambda i: (0, 0)),
                  pl.BlockSpec((1, 128), lambda i: (0, 0))],
        out_specs=[pl.BlockSpec((EBLK, 32), lambda i: (i, 0)),
                   pl.BlockSpec((8, 32), lambda i: (0, 0)),
                   pl.BlockSpec((8, 32), lambda i: (0, 0))],
        out_shape=[_sds((E, 32)), _sds((8, 32)), _sds((8, 32))],
    )(g1i, g2i, wm2, bm2, we1k, beta_e, we2p, bm1, bnir)


def _tc_edge_softmax(z, m, s):
    def body(z_r, m_r, s_r, out_o):
        val = jnp.exp(z_r[...] - m_r[0:1, :]) / s_r[0:1, :]
        out_o[...] = val[:, :27]

    return pl.pallas_call(
        body,
        grid=(NEB,),
        in_specs=[pl.BlockSpec((EBLK, 32), lambda i: (i, 0)),
                  pl.BlockSpec((8, 32), lambda i: (0, 0)),
                  pl.BlockSpec((8, 32), lambda i: (0, 0))],
        out_specs=pl.BlockSpec((EBLK, 27), lambda i: (i, 0)),
        out_shape=_sds((E, 27)),
    )(z, m, s)


def kernel(node_feats, edge_index, Wg1, bg1, Wg2, bg2, Wea, bea, Wna, bna,
           Wnir, bnir, Wm1, bm1, Wm2, bm2, Wn1, gamma_n, beta_n, Wn2,
           We1, gamma_e, beta_e, We2):
    x = node_feats
    subj = edge_index[:, 0]
    obj = edge_index[:, 1]

    wa = Wea[:128] - Wea[128:]
    wb = Wea[128:]
    wu = Wm1[:128] - Wm1[128:]
    wv = Wm1[128:]
    wpq = jnp.concatenate([Wnir[:128], Wnir[128:]], axis=1)
    kn = gamma_n / jnp.sqrt(1.0 + 1e-5)
    wn1k = Wn1 * kn[None, :]
    ke = gamma_e / jnp.sqrt(1.0 + 1e-5)
    we1k = We1 * ke[None, :]
    we2p = jnp.concatenate([We2, jnp.zeros((128, 5), _F32)], axis=1)
    r = lambda v: v.reshape(1, -1)
    zeros128 = jnp.zeros((SLAB, 128), _F32)

    aaug, baug, h1, u, v = _tc_proj(x, wa, wb, Wg1, wu, wv)
    s1p = _sc_segment_sum(baug, obj, subj, zeros128)
    s2p = _sc_segment_sum(aaug, subj, obj, zeros128)
    g1, agg = _tc_attn_combine(s1p[0], s1p[1], s2p[0], s2p[1],
                               aaug, baug, h1, r(bea))
    m1p = _sc_segment_sum(g1, subj, obj, zeros128)
    g2 = _tc_gcn1(m1p[0], m1p[1], g1, agg, s2p[0], s2p[1], Wg2, r(bg1))
    m2p = _sc_segment_sum(g2, subj, obj, zeros128)
    upi, vqi, node_logits = _tc_gcn2_heads(
        m2p[0], m2p[1], g2, s2p[0], s2p[1], u, v,
        Wna, r(bna), wpq, None, wn1k, r(beta_n), Wn2, r(bg2))
    g1i, g2i = _sc_edge_gather(upi, vqi, subj, obj)
    z, m, s = _tc_edge_head(g1i, g2i, Wm2, r(bm2), we1k, r(beta_e),
                            we2p, r(bm1), r(bnir))
    edge_logits = _tc_edge_softmax(z, m, s)
    return node_logits, edge_logits


# speedup vs baseline: 9.2012x; 1.3632x over previous
import functools

import jax
import jax.numpy as jnp
from jax import lax
from jax.experimental import pallas as pl
from jax.experimental.pallas import tpu as pltpu
from jax.experimental.pallas import tpu_sc as plsc

N = 10000
E = 320000
NC = 2
NS = 16
NW = NC * NS
EW = E // NW
CH = 80
NCHUNK = EW // CH
SLOTS = 5
NGRP = NCHUNK // SLOTS
CHS = 40
NCHS = EW // CHS
NGRPS = NCHS // SLOTS
EW2 = E // NS
NCHS2 = EW2 // CHS
NGRPS2 = NCHS2 // SLOTS
NPAD = 10240
SLAB = NPAD // NS

RB = 2000
NB = N // RB
HEA = 166400
HEB = E - HEA
EBLK = 1280

_F32 = jnp.float32
_MASK_HI = -65536


def _sds(shape, dtype=_F32):
    return jax.ShapeDtypeStruct(shape, dtype)


def _sigmoid(x):
    return 1.0 / (1.0 + jnp.exp(-x))


def _bf16_bits(x):
    b = lax.bitcast_convert_type(x, jnp.int32)
    return b + 0x7FFF + (lax.shift_right_logical(b, 16) & 1)


def _pack2(hi, lo):
    return (_bf16_bits(hi) & _MASK_HI) | lax.shift_right_logical(_bf16_bits(lo), 16)


def _unpack_hi(w):
    return lax.bitcast_convert_type(w & _MASK_HI, _F32)


def _unpack_lo(w):
    return lax.bitcast_convert_type(jnp.left_shift(w, 16), _F32)


def _mesh():
    return plsc.VectorSubcoreMesh(
        core_axis_name="c", subcore_axis_name="s", num_cores=NC, num_subcores=NS)


def _sc_attn_pair(baug, aaug, subj, obj, zeros):
    @functools.partial(
        pl.kernel,
        out_type=[_sds((NPAD, 128)), _sds((NPAD, 128))],
        mesh=_mesh(),
        scratch_types=[
            pltpu.VMEM((EW2,), jnp.int32),
            pltpu.VMEM((SLOTS, CHS), jnp.int32),
            pltpu.VMEM((SLOTS, CHS, 128), _F32),
            pltpu.VMEM_SHARED((NPAD, 128), _F32),
        ] + [pltpu.SemaphoreType.DMA] * (2 * SLOTS),
    )
    def body(b_hbm, a_hbm, s_hbm, o_hbm, z_hbm, s1_hbm, s2_hbm,
             gflat, sidx, rows, acc, *sems):
        gsem = sems[:SLOTS]
        ssem = sems[SLOTS:]
        cid = lax.axis_index("c")
        sid = lax.axis_index("s")
        base = sid * EW2
        pltpu.sync_copy(z_hbm, acc.at[pl.ds(sid * SLAB, SLAB), :])

        def run(tab_hbm, gsrc_hbm, ssrc_hbm, out_hbm):
            pltpu.sync_copy(gsrc_hbm.at[pl.ds(base, EW2)], gflat)
            plsc.subcore_barrier()

            def group(g, carry):
                c0 = g * SLOTS
                descs = []
                for k in range(SLOTS):
                    @pl.when(g > 0)
                    def _(k=k):
                        pltpu.make_async_copy(
                            rows.at[k], acc.at[sidx.at[k]], ssem[k]).wait()
                    pltpu.sync_copy(
                        ssrc_hbm.at[pl.ds(base + (c0 + k) * CHS, CHS)],
                        sidx.at[k])
                    descs.append(
                        pltpu.async_copy(
                            tab_hbm.at[gflat.at[pl.ds((c0 + k) * CHS, CHS)]],
                            rows.at[k], gsem[k]))
                for k in range(SLOTS):
                    descs[k].wait()
                    pltpu.async_copy(rows.at[k], acc.at[sidx.at[k]], ssem[k],
                                     add=True)
                return carry

            lax.fori_loop(0, NGRPS2, group, 0)
            for k in range(SLOTS):
                pltpu.make_async_copy(
                    rows.at[k], acc.at[sidx.at[k]], ssem[k]).wait()
            plsc.subcore_barrier()
            sl = pl.ds(sid * SLAB, SLAB)
            pltpu.sync_copy(acc.at[sl, :], out_hbm.at[sl, :])

        @pl.when(cid == 0)
        def _():
            run(b_hbm, o_hbm, s_hbm, s1_hbm)

        @pl.when(cid == 1)
        def _():
            run(a_hbm, s_hbm, o_hbm, s2_hbm)

    return body(baug, aaug, subj, obj, zeros)


def _sc_segment_sum(table, gather_idx, scatter_idx, zeros):
    @functools.partial(
        pl.kernel,
        out_type=_sds((NC, NPAD, 128)),
        mesh=_mesh(),
        scratch_types=[
            pltpu.VMEM((EW,), jnp.int32),
            pltpu.VMEM((SLOTS, CHS), jnp.int32),
            pltpu.VMEM((SLOTS, CHS, 128), _F32),
            pltpu.VMEM_SHARED((NPAD, 128), _F32),
        ] + [pltpu.SemaphoreType.DMA] * (2 * SLOTS),
    )
    def body(tab_hbm, g_hbm, s_hbm, z_hbm, out_hbm, gflat, sidx, rows, acc, *sems):
        gsem = sems[:SLOTS]
        ssem = sems[SLOTS:]
        cid = lax.axis_index("c")
        sid = lax.axis_index("s")
        wid = cid * NS + sid
        base = wid * EW
        pltpu.sync_copy(z_hbm, acc.at[pl.ds(sid * SLAB, SLAB), :])
        pltpu.sync_copy(g_hbm.at[pl.ds(base, EW)], gflat)
        plsc.subcore_barrier()

        def group(g, carry):
            c0 = g * SLOTS
            descs = []
            for k in range(SLOTS):
                @pl.when(g > 0)
                def _(k=k):
                    pltpu.make_async_copy(
                        rows.at[k], acc.at[sidx.at[k]], ssem[k]).wait()
                pltpu.sync_copy(
                    s_hbm.at[pl.ds(base + (c0 + k) * CHS, CHS)], sidx.at[k])
                descs.append(
                    pltpu.async_copy(
                        tab_hbm.at[gflat.at[pl.ds((c0 + k) * CHS, CHS)]],
                        rows.at[k], gsem[k]))
            for k in range(SLOTS):
                descs[k].wait()
                pltpu.async_copy(rows.at[k], acc.at[sidx.at[k]], ssem[k],
                                 add=True)
            return carry

        lax.fori_loop(0, NGRPS, group, 0)
        for k in range(SLOTS):
            pltpu.make_async_copy(rows.at[k], acc.at[sidx.at[k]], ssem[k]).wait()
        plsc.subcore_barrier()
        sl = pl.ds(sid * SLAB, SLAB)
        pltpu.sync_copy(acc.at[sl, :], out_hbm.at[cid, sl, :])

    return body(table, gather_idx, scatter_idx, zeros)


def _sc_edge_gather(upi, vqi, subj, obj, start, he):
    ewh = he // NW
    nch = ewh // CH
    ngrp = nch // SLOTS

    @functools.partial(
        pl.kernel,
        out_type=[_sds((he, 128), jnp.int32), _sds((he, 128), jnp.int32)],
        mesh=_mesh(),
        scratch_types=[
            pltpu.VMEM((ewh,), jnp.int32),
            pltpu.VMEM((ewh,), jnp.int32),
            pltpu.VMEM((SLOTS, CH, 128), jnp.int32),
            pltpu.VMEM((SLOTS, CH, 128), jnp.int32),
        ] + [pltpu.SemaphoreType.DMA] * (4 * SLOTS),
    )
    def body(up_hbm, vq_hbm, s_hbm, o_hbm, g1_hbm, g2_hbm,
             sflat, oflat, r1, r2, *sems):
        gs1 = sems[:SLOTS]
        gs2 = sems[SLOTS:2 * SLOTS]
        ws1 = sems[2 * SLOTS:3 * SLOTS]
        ws2 = sems[3 * SLOTS:]
        cid = lax.axis_index("c")
        sid = lax.axis_index("s")
        wid = cid * NS + sid
        base = wid * ewh
        pltpu.sync_copy(s_hbm.at[pl.ds(start + base, ewh)], sflat)
        pltpu.sync_copy(o_hbm.at[pl.ds(start + base, ewh)], oflat)

        def group(g, carry):
            c0 = g * SLOTS
            descs = []
            for k in range(SLOTS):
                @pl.when(g > 0)
                def _(k=k, c0=c0):
                    offp = base + (c0 - SLOTS + k) * CH
                    pltpu.make_async_copy(
                        r1.at[k], g1_hbm.at[pl.ds(offp, CH), :], ws1[k]).wait()
                    pltpu.make_async_copy(
                        r2.at[k], g2_hbm.at[pl.ds(offp, CH), :], ws2[k]).wait()
                sl = pl.ds((c0 + k) * CH, CH)
                descs.append(
                    pltpu.async_copy(up_hbm.at[sflat.at[sl]], r1.at[k], gs1[k]))
                descs.append(
                    pltpu.async_copy(vq_hbm.at[oflat.at[sl]], r2.at[k], gs2[k]))
            for k in range(SLOTS):
                off = base + (c0 + k) * CH
                descs[2 * k].wait()
                pltpu.async_copy(r1.at[k], g1_hbm.at[pl.ds(off, CH), :], ws1[k])
                descs[2 * k + 1].wait()
                pltpu.async_copy(r2.at[k], g2_hbm.at[pl.ds(off, CH), :], ws2[k])
            return carry

        lax.fori_loop(0, ngrp, group, 0)
        for k in range(SLOTS):
            offp = base + ((ngrp - 1) * SLOTS + k) * CH
            pltpu.make_async_copy(
                r1.at[k], g1_hbm.at[pl.ds(offp, CH), :], ws1[k]).wait()
            pltpu.make_async_copy(
                r2.at[k], g2_hbm.at[pl.ds(offp, CH), :], ws2[k]).wait()

    return body(upi, vqi, subj, obj)


def _dot(a, b):
    return jnp.dot(a, b, preferred_element_type=_F32)


def _tc_proj(x, wa, wb, wg1, wu, wv):
    def body(x_r, wa_r, wb_r, wg_r, wu_r, wv_r, aaug_o, baug_o, h1_o, u_o, v_o):
        xb = x_r[...]
        ones = jnp.ones((RB, 64), _F32)
        aaug_o[...] = jnp.concatenate([_dot(xb, wa_r[...]), ones], axis=1)
        baug_o[...] = jnp.concatenate([_dot(xb, wb_r[...]), ones], axis=1)
        h1_o[...] = _dot(xb, wg_r[...])
        u_o[...] = _dot(xb, wu_r[...])
        v_o[...] = _dot(xb, wv_r[...])

    return pl.pallas_call(
        body,
        grid=(NB,),
        in_specs=[
            pl.BlockSpec((RB, 128), lambda i: (i, 0)),
            pl.BlockSpec((128, 64), lambda i: (0, 0)),
            pl.BlockSpec((128, 64), lambda i: (0, 0)),
            pl.BlockSpec((128, 64), lambda i: (0, 0)),
            pl.BlockSpec((128, 128), lambda i: (0, 0)),
            pl.BlockSpec((128, 128), lambda i: (0, 0)),
        ],
        out_specs=[
            pl.BlockSpec((RB, 128), lambda i: (i, 0)),
            pl.BlockSpec((RB, 128), lambda i: (i, 0)),
            pl.BlockSpec((RB, 64), lambda i: (i, 0)),
            pl.BlockSpec((RB, 128), lambda i: (i, 0)),
            pl.BlockSpec((RB, 128), lambda i: (i, 0)),
        ],
        out_shape=[_sds((N, 128)), _sds((N, 128)), _sds((N, 64)),
                   _sds((N, 128)), _sds((N, 128))],
    )(x, wa, wb, wg1, wu, wv)


def _tc_attn_combine(s1, s2, aaug, baug, h1, bea):
    def body(s1_r, s2_r, a_r, b_r, h1_r, bea_r, g1_o, agg_o):
        s1 = s1_r[...]
        s2 = s2_r[...]
        a = a_r[...][:, :64]
        b = b_r[...][:, :64]
        cs = s1[:, 64:65]
        co = s2[:, 64:65]
        bea_v = bea_r[...]
        mean_s = (cs * (a + bea_v) + s1[:, :64]) / jnp.maximum(cs, 1.0)
        mean_o = (s2[:, :64] + co * (b + bea_v)) / jnp.maximum(co, 1.0)
        agg_o[...] = _sigmoid(mean_s * mean_o)
        dis = lax.rsqrt(1.0 + co)
        g1_o[...] = jnp.concatenate(
            [h1_r[...] * dis, jnp.zeros((RB, 64), _F32)], axis=1)

    spec128 = pl.BlockSpec((RB, 128), lambda i: (i, 0))
    spec64 = pl.BlockSpec((RB, 64), lambda i: (i, 0))
    return pl.pallas_call(
        body,
        grid=(NB,),
        in_specs=[spec128, spec128, spec128, spec128, spec64,
                  pl.BlockSpec((1, 64), lambda i: (0, 0))],
        out_specs=[spec128, spec64],
        out_shape=[_sds((N, 128)), _sds((N, 64))],
    )(s1, s2, aaug, baug, h1, bea)


def _tc_gcn1(m1a, m1b, g1, agg, s2, wg2, bg1):
    def body(m1a_r, m1b_r, g1_r, agg_r, s2_r, wg2_r, bg1_r, g2_o):
        co = s2_r[...][:, 64:65]
        dis = lax.rsqrt(1.0 + co)
        m1 = (m1a_r[...] + m1b_r[...])[:, :64]
        g1v = g1_r[...][:, :64]
        h = jnp.maximum(dis * (m1 + g1v) + bg1_r[...], 0.0) * agg_r[...]
        g2_o[...] = _dot(h, wg2_r[...]) * dis

    spec128 = pl.BlockSpec((RB, 128), lambda i: (i, 0))
    spec64 = pl.BlockSpec((RB, 64), lambda i: (i, 0))
    return pl.pallas_call(
        body,
        grid=(NB,),
        in_specs=[spec128, spec128, spec128, spec64, spec128,
                  pl.BlockSpec((64, 128), lambda i: (0, 0)),
                  pl.BlockSpec((1, 64), lambda i: (0, 0))],
        out_specs=pl.BlockSpec((RB, 128), lambda i: (i, 0)),
        out_shape=_sds((N, 128)),
    )(m1a, m1b, g1, agg, s2, wg2, bg1)


def _tc_gcn2_pack(m2a, m2b, g2, s2, u, v, wna, bna, wpq, bg2):
    def body(m2a_r, m2b_r, g2_r, s2_r, u_r, v_r,
             wna_r, bna_r, wpq_r, bg2_r, upi_o, vqi_o, h2_o):
        co = s2_r[...][:, 64:65]
        dis = lax.rsqrt(1.0 + co)
        h2 = jnp.maximum(dis * (m2a_r[...] + m2b_r[...] + g2_r[...]) + bg2_r[...], 0.0)
        ni = jnp.maximum(_dot(h2, wna_r[...]) + bna_r[...], 0.0)
        pq = _dot(ni, wpq_r[...])
        upi_o[...] = _pack2(u_r[...], pq[:, :128])
        vqi_o[...] = _pack2(v_r[...], pq[:, 128:])
        h2_o[...] = h2

    spec128 = pl.BlockSpec((RB, 128), lambda i: (i, 0))
    return pl.pallas_call(
        body,
        grid=(NB,),
        in_specs=[spec128, spec128, spec128, spec128, spec128, spec128,
                  pl.BlockSpec((128, 128), lambda i: (0, 0)),
                  pl.BlockSpec((1, 128), lambda i: (0, 0)),
                  pl.BlockSpec((128, 256), lambda i: (0, 0)),
                  pl.BlockSpec((1, 128), lambda i: (0, 0))],
        out_specs=[spec128, spec128, spec128],
        out_shape=[_sds((N, 128), jnp.int32), _sds((N, 128), jnp.int32),
                   _sds((N, 128))],
    )(m2a, m2b, g2, s2, u, v, wna, bna, wpq, bg2)


def _tc_node_head(h2, wn1k, beta_n, wn2):
    def body(h2_r, wn1k_r, bn_r, wn2_r, nl_o):
        nx = _dot(h2_r[...], wn1k_r[...]) + bn_r[...]
        nx = jnp.where(nx > 0, nx, 0.2 * nx)
        logits = _dot(nx, wn2_r[...])
        logits = logits - jnp.max(logits, axis=1, keepdims=True)
        el = jnp.exp(logits)
        nl_o[...] = el / jnp.sum(el, axis=1, keepdims=True)

    return pl.pallas_call(
        body,
        grid=(NB,),
        in_specs=[pl.BlockSpec((RB, 128), lambda i: (i, 0)),
                  pl.BlockSpec((128, 64), lambda i: (0, 0)),
                  pl.BlockSpec((1, 64), lambda i: (0, 0)),
                  pl.BlockSpec((64, 160), lambda i: (0, 0))],
        out_specs=pl.BlockSpec((RB, 160), lambda i: (i, 0)),
        out_shape=_sds((N, 160)),
    )(h2, wn1k, beta_n, wn2)


def _tc_edge_head(g1i, g2i, wm2, bm2, we1k, beta_e, we2p, bm1, bnir,
                  m_init, s_init, he):
    neb = he // EBLK

    def body(g1_r, g2_r, wm2_r, bm2_r, we1_r, be_r, we2_r, bm1_r, bnir_r,
             mi_r, si_r, z_o, m_o, s_o):
        i = pl.program_id(0)
        w1 = g1_r[...]
        w2 = g2_r[...]
        t_pre = _unpack_hi(w1) + _unpack_hi(w2) + bm1_r[...]
        a_pre = _unpack_lo(w1) + _unpack_lo(w2) + bnir_r[...]
        t = jnp.maximum(t_pre, 0.0) * _sigmoid(a_pre)
        ef = jnp.maximum(_dot(t, wm2_r[...]) + bm2_r[...], 0.0)
        ex = _dot(ef, we1_r[...]) + be_r[...]
        ex = jnp.where(ex > 0, ex, 0.2 * ex)
        z = _dot(ex, we2_r[...])
        z_o[...] = z

        @pl.when(i == 0)
        def _():
            m_o[...] = mi_r[...]
            s_o[...] = si_r[...]

        bmax = jnp.max(z, axis=0, keepdims=True)
        m_old = m_o[0:1, :]
        s_old = s_o[0:1, :]
        m_new = jnp.maximum(m_old, bmax)
        s_new = s_old * jnp.exp(m_old - m_new) + jnp.sum(
            jnp.exp(z - m_new), axis=0, keepdims=True)
        m_o[...] = jnp.broadcast_to(m_new, (8, 32))
        s_o[...] = jnp.broadcast_to(s_new, (8, 32))

    speci = pl.BlockSpec((EBLK, 128), lambda i: (i, 0))
    spec_ms = pl.BlockSpec((8, 32), lambda i: (0, 0))
    return pl.pallas_call(
        body,
        grid=(neb,),
        in_specs=[speci, speci,
                  pl.BlockSpec((128, 256), lambda i: (0, 0)),
                  pl.BlockSpec((1, 256), lambda i: (0, 0)),
                  pl.BlockSpec((256, 128), lambda i: (0, 0)),
                  pl.BlockSpec((1, 128), lambda i: (0, 0)),
                  pl.BlockSpec((128, 32), lambda i: (0, 0)),
                  pl.BlockSpec((1, 128), lambda i: (0, 0)),
                  pl.BlockSpec((1, 128), lambda i: (0, 0)),
                  spec_ms, spec_ms],
        out_specs=[pl.BlockSpec((EBLK, 32), lambda i: (i, 0)),
                   spec_ms, spec_ms],
        out_shape=[_sds((he, 32)), _sds((8, 32)), _sds((8, 32))],
    )(g1i, g2i, wm2, bm2, we1k, beta_e, we2p, bm1, bnir, m_init, s_init)


def _tc_edge_softmax(z, m, s, he):
    sblk = 2 * EBLK

    def body(z_r, m_r, s_r, out_o):
        val = jnp.exp(z_r[...] - m_r[0:1, :]) / s_r[0:1, :]
        out_o[...] = val[:, :27]

    return pl.pallas_call(
        body,
        grid=(he // sblk,),
        in_specs=[pl.BlockSpec((sblk, 32), lambda i: (i, 0)),
                  pl.BlockSpec((8, 32), lambda i: (0, 0)),
                  pl.BlockSpec((8, 32), lambda i: (0, 0))],
        out_specs=pl.BlockSpec((sblk, 27), lambda i: (i, 0)),
        out_shape=_sds((he, 27)),
    )(z, m, s)


def kernel(node_feats, edge_index, Wg1, bg1, Wg2, bg2, Wea, bea, Wna, bna,
           Wnir, bnir, Wm1, bm1, Wm2, bm2, Wn1, gamma_n, beta_n, Wn2,
           We1, gamma_e, beta_e, We2):
    x = node_feats
    subj = edge_index[:, 0]
    obj = edge_index[:, 1]

    wa = Wea[:128] - Wea[128:]
    wb = Wea[128:]
    wu = Wm1[:128] - Wm1[128:]
    wv = Wm1[128:]
    wpq = jnp.concatenate([Wnir[:128], Wnir[128:]], axis=1)
    kn = gamma_n / jnp.sqrt(1.0 + 1e-5)
    wn1k = Wn1 * kn[None, :]
    ke = gamma_e / jnp.sqrt(1.0 + 1e-5)
    we1k = We1 * ke[None, :]
    we2p = jnp.concatenate([We2, jnp.zeros((128, 5), _F32)], axis=1)
    r = lambda v: v.reshape(1, -1)
    zeros128 = jnp.zeros((SLAB, 128), _F32)

    aaug, baug, h1, u, v = _tc_proj(x, wa, wb, Wg1, wu, wv)
    s1, s2 = _sc_attn_pair(baug, aaug, subj, obj, zeros128)
    g1, agg = _tc_attn_combine(s1, s2, aaug, baug, h1, r(bea))
    m1p = _sc_segment_sum(g1, subj, obj, zeros128)
    g2 = _tc_gcn1(m1p[0], m1p[1], g1, agg, s2, Wg2, r(bg1))
    m2p = _sc_segment_sum(g2, subj, obj, zeros128)
    upi, vqi, h2 = _tc_gcn2_pack(m2p[0], m2p[1], g2, s2, u, v,
                                 Wna, r(bna), wpq, r(bg2))
    m_init = jnp.full((8, 32), -1e30, _F32)
    s_init = jnp.zeros((8, 32), _F32)
    g1a, g2a = _sc_edge_gather(upi, vqi, subj, obj, 0, HEA)
    g1b, g2b = _sc_edge_gather(upi, vqi, subj, obj, HEA, HEB)
    za, ma, sa = _tc_edge_head(g1a, g2a, Wm2, r(bm2), we1k, r(beta_e),
                               we2p, r(bm1), r(bnir), m_init, s_init, HEA)
    node_logits = _tc_node_head(h2, wn1k, r(beta_n), Wn2)
    zb, m, s = _tc_edge_head(g1b, g2b, Wm2, r(bm2), we1k, r(beta_e),
                             we2p, r(bm1), r(bnir), ma, sa, HEB)
    ea = _tc_edge_softmax(za, m, s, HEA)
    eb = _tc_edge_softmax(zb, m, s, HEB)
    edge_logits = jnp.concatenate([ea, eb], axis=0)
    return node_logits, edge_logits


# speedup vs baseline: 9.7057x; 1.0548x over previous
import functools

import jax
import jax.numpy as jnp
from jax import lax
from jax.experimental import pallas as pl
from jax.experimental.pallas import tpu as pltpu
from jax.experimental.pallas import tpu_sc as plsc

N = 10000
E = 320000
NC = 2
NS = 16
NW = NC * NS
EW = E // NW
CH = 80
NCHUNK = EW // CH
SLOTS = 5
NGRP = NCHUNK // SLOTS
CHS = 40
NCHS = EW // CHS
NGRPS = NCHS // SLOTS
EW2 = E // NS
NCHS2 = EW2 // CHS
NGRPS2 = NCHS2 // SLOTS
NPAD = 10240
SLAB = NPAD // NS

RB = 2000
NB = N // RB
HEA = 166400
HEB = E - HEA
EBLK = 2560

_F32 = jnp.float32
_MASK_HI = -65536


def _sds(shape, dtype=_F32):
    return jax.ShapeDtypeStruct(shape, dtype)


def _sigmoid(x):
    return 1.0 / (1.0 + jnp.exp(-x))


def _bf16_bits(x):
    b = lax.bitcast_convert_type(x, jnp.int32)
    return b + 0x7FFF + (lax.shift_right_logical(b, 16) & 1)


def _pack2(hi, lo):
    return (_bf16_bits(hi) & _MASK_HI) | lax.shift_right_logical(_bf16_bits(lo), 16)


def _unpack_hi(w):
    return lax.bitcast_convert_type(w & _MASK_HI, _F32)


def _unpack_lo(w):
    return lax.bitcast_convert_type(jnp.left_shift(w, 16), _F32)


def _mesh():
    return plsc.VectorSubcoreMesh(
        core_axis_name="c", subcore_axis_name="s", num_cores=NC, num_subcores=NS)


def _sc_attn_pair(baug, aaug, subj, obj, zeros):
    @functools.partial(
        pl.kernel,
        out_type=[_sds((NPAD, 128)), _sds((NPAD, 128))],
        mesh=_mesh(),
        scratch_types=[
            pltpu.VMEM((EW2,), jnp.int32),
            pltpu.VMEM((SLOTS, CHS), jnp.int32),
            pltpu.VMEM((SLOTS, CHS, 128), _F32),
            pltpu.VMEM_SHARED((NPAD, 128), _F32),
        ] + [pltpu.SemaphoreType.DMA] * (2 * SLOTS),
    )
    def body(b_hbm, a_hbm, s_hbm, o_hbm, z_hbm, s1_hbm, s2_hbm,
             gflat, sidx, rows, acc, *sems):
        gsem = sems[:SLOTS]
        ssem = sems[SLOTS:]
        cid = lax.axis_index("c")
        sid = lax.axis_index("s")
        base = sid * EW2
        pltpu.sync_copy(z_hbm, acc.at[pl.ds(sid * SLAB, SLAB), :])

        def run(tab_hbm, gsrc_hbm, ssrc_hbm, out_hbm):
            pltpu.sync_copy(gsrc_hbm.at[pl.ds(base, EW2)], gflat)
            plsc.subcore_barrier()

            def group(g, carry):
                c0 = g * SLOTS
                descs = []
                for k in range(SLOTS):
                    @pl.when(g > 0)
                    def _(k=k):
                        pltpu.make_async_copy(
                            rows.at[k], acc.at[sidx.at[k]], ssem[k]).wait()
                    pltpu.sync_copy(
                        ssrc_hbm.at[pl.ds(base + (c0 + k) * CHS, CHS)],
                        sidx.at[k])
                    descs.append(
                        pltpu.async_copy(
                            tab_hbm.at[gflat.at[pl.ds((c0 + k) * CHS, CHS)]],
                            rows.at[k], gsem[k]))
                for k in range(SLOTS):
                    descs[k].wait()
                    pltpu.async_copy(rows.at[k], acc.at[sidx.at[k]], ssem[k],
                                     add=True)
                return carry

            lax.fori_loop(0, NGRPS2, group, 0)
            for k in range(SLOTS):
                pltpu.make_async_copy(
                    rows.at[k], acc.at[sidx.at[k]], ssem[k]).wait()
            plsc.subcore_barrier()
            sl = pl.ds(sid * SLAB, SLAB)
            pltpu.sync_copy(acc.at[sl, :], out_hbm.at[sl, :])

        @pl.when(cid == 0)
        def _():
            run(b_hbm, o_hbm, s_hbm, s1_hbm)

        @pl.when(cid == 1)
        def _():
            run(a_hbm, s_hbm, o_hbm, s2_hbm)

    return body(baug, aaug, subj, obj, zeros)


def _sc_segment_sum(table, gather_idx, scatter_idx, zeros):
    @functools.partial(
        pl.kernel,
        out_type=_sds((NC, NPAD, 128)),
        mesh=_mesh(),
        scratch_types=[
            pltpu.VMEM((EW,), jnp.int32),
            pltpu.VMEM((SLOTS, CHS), jnp.int32),
            pltpu.VMEM((SLOTS, CHS, 128), _F32),
            pltpu.VMEM_SHARED((NPAD, 128), _F32),
        ] + [pltpu.SemaphoreType.DMA] * (2 * SLOTS),
    )
    def body(tab_hbm, g_hbm, s_hbm, z_hbm, out_hbm, gflat, sidx, rows, acc, *sems):
        gsem = sems[:SLOTS]
        ssem = sems[SLOTS:]
        cid = lax.axis_index("c")
        sid = lax.axis_index("s")
        wid = cid * NS + sid
        base = wid * EW
        pltpu.sync_copy(z_hbm, acc.at[pl.ds(sid * SLAB, SLAB), :])
        pltpu.sync_copy(g_hbm.at[pl.ds(base, EW)], gflat)
        plsc.subcore_barrier()

        def group(g, carry):
            c0 = g * SLOTS
            descs = []
            for k in range(SLOTS):
                @pl.when(g > 0)
                def _(k=k):
                    pltpu.make_async_copy(
                        rows.at[k], acc.at[sidx.at[k]], ssem[k]).wait()
                pltpu.sync_copy(
                    s_hbm.at[pl.ds(base + (c0 + k) * CHS, CHS)], sidx.at[k])
                descs.append(
                    pltpu.async_copy(
                        tab_hbm.at[gflat.at[pl.ds((c0 + k) * CHS, CHS)]],
                        rows.at[k], gsem[k]))
            for k in range(SLOTS):
                descs[k].wait()
                pltpu.async_copy(rows.at[k], acc.at[sidx.at[k]], ssem[k],
                                 add=True)
            return carry

        lax.fori_loop(0, NGRPS, group, 0)
        for k in range(SLOTS):
            pltpu.make_async_copy(rows.at[k], acc.at[sidx.at[k]], ssem[k]).wait()
        plsc.subcore_barrier()
        sl = pl.ds(sid * SLAB, SLAB)
        pltpu.sync_copy(acc.at[sl, :], out_hbm.at[cid, sl, :])

    return body(table, gather_idx, scatter_idx, zeros)


def _sc_edge_gather(upi, vqi, subj, obj, start, he):
    ewh = he // NW
    nch = ewh // CH
    ngrp = nch // SLOTS

    @functools.partial(
        pl.kernel,
        out_type=[_sds((he, 128), jnp.int32), _sds((he, 128), jnp.int32)],
        mesh=_mesh(),
        scratch_types=[
            pltpu.VMEM((ewh,), jnp.int32),
            pltpu.VMEM((ewh,), jnp.int32),
            pltpu.VMEM((SLOTS, CH, 128), jnp.int32),
            pltpu.VMEM((SLOTS, CH, 128), jnp.int32),
        ] + [pltpu.SemaphoreType.DMA] * (4 * SLOTS),
    )
    def body(up_hbm, vq_hbm, s_hbm, o_hbm, g1_hbm, g2_hbm,
             sflat, oflat, r1, r2, *sems):
        gs1 = sems[:SLOTS]
        gs2 = sems[SLOTS:2 * SLOTS]
        ws1 = sems[2 * SLOTS:3 * SLOTS]
        ws2 = sems[3 * SLOTS:]
        cid = lax.axis_index("c")
        sid = lax.axis_index("s")
        wid = cid * NS + sid
        base = wid * ewh
        pltpu.sync_copy(s_hbm.at[pl.ds(start + base, ewh)], sflat)
        pltpu.sync_copy(o_hbm.at[pl.ds(start + base, ewh)], oflat)

        def group(g, carry):
            c0 = g * SLOTS
            descs = []
            for k in range(SLOTS):
                @pl.when(g > 0)
                def _(k=k, c0=c0):
                    offp = base + (c0 - SLOTS + k) * CH
                    pltpu.make_async_copy(
                        r1.at[k], g1_hbm.at[pl.ds(offp, CH), :], ws1[k]).wait()
                    pltpu.make_async_copy(
                        r2.at[k], g2_hbm.at[pl.ds(offp, CH), :], ws2[k]).wait()
                sl = pl.ds((c0 + k) * CH, CH)
                descs.append(
                    pltpu.async_copy(up_hbm.at[sflat.at[sl]], r1.at[k], gs1[k]))
                descs.append(
                    pltpu.async_copy(vq_hbm.at[oflat.at[sl]], r2.at[k], gs2[k]))
            for k in range(SLOTS):
                off = base + (c0 + k) * CH
                descs[2 * k].wait()
                pltpu.async_copy(r1.at[k], g1_hbm.at[pl.ds(off, CH), :], ws1[k])
                descs[2 * k + 1].wait()
                pltpu.async_copy(r2.at[k], g2_hbm.at[pl.ds(off, CH), :], ws2[k])
            return carry

        lax.fori_loop(0, ngrp, group, 0)
        for k in range(SLOTS):
            offp = base + ((ngrp - 1) * SLOTS + k) * CH
            pltpu.make_async_copy(
                r1.at[k], g1_hbm.at[pl.ds(offp, CH), :], ws1[k]).wait()
            pltpu.make_async_copy(
                r2.at[k], g2_hbm.at[pl.ds(offp, CH), :], ws2[k]).wait()

    return body(upi, vqi, subj, obj)


def _dot(a, b):
    return jnp.dot(a, b, preferred_element_type=_F32)


def _tc_proj(x, wa, wb, wg1, wu, wv):
    def body(x_r, wa_r, wb_r, wg_r, wu_r, wv_r, aaug_o, baug_o, h1_o, u_o, v_o):
        xb = x_r[...]
        ones = jnp.ones((RB, 64), _F32)
        aaug_o[...] = jnp.concatenate([_dot(xb, wa_r[...]), ones], axis=1)
        baug_o[...] = jnp.concatenate([_dot(xb, wb_r[...]), ones], axis=1)
        h1_o[...] = _dot(xb, wg_r[...])
        u_o[...] = _dot(xb, wu_r[...])
        v_o[...] = _dot(xb, wv_r[...])

    return pl.pallas_call(
        body,
        grid=(NB,),
        in_specs=[
            pl.BlockSpec((RB, 128), lambda i: (i, 0)),
            pl.BlockSpec((128, 64), lambda i: (0, 0)),
            pl.BlockSpec((128, 64), lambda i: (0, 0)),
            pl.BlockSpec((128, 64), lambda i: (0, 0)),
            pl.BlockSpec((128, 128), lambda i: (0, 0)),
            pl.BlockSpec((128, 128), lambda i: (0, 0)),
        ],
        out_specs=[
            pl.BlockSpec((RB, 128), lambda i: (i, 0)),
            pl.BlockSpec((RB, 128), lambda i: (i, 0)),
            pl.BlockSpec((RB, 64), lambda i: (i, 0)),
            pl.BlockSpec((RB, 128), lambda i: (i, 0)),
            pl.BlockSpec((RB, 128), lambda i: (i, 0)),
        ],
        out_shape=[_sds((N, 128)), _sds((N, 128)), _sds((N, 64)),
                   _sds((N, 128)), _sds((N, 128))],
    )(x, wa, wb, wg1, wu, wv)


def _tc_attn_combine(s1, s2, aaug, baug, h1, bea):
    def body(s1_r, s2_r, a_r, b_r, h1_r, bea_r, g1_o, agg_o):
        s1 = s1_r[...]
        s2 = s2_r[...]
        a = a_r[...][:, :64]
        b = b_r[...][:, :64]
        cs = s1[:, 64:65]
        co = s2[:, 64:65]
        bea_v = bea_r[...]
        mean_s = (cs * (a + bea_v) + s1[:, :64]) / jnp.maximum(cs, 1.0)
        mean_o = (s2[:, :64] + co * (b + bea_v)) / jnp.maximum(co, 1.0)
        agg_o[...] = _sigmoid(mean_s * mean_o)
        dis = lax.rsqrt(1.0 + co)
        g1_o[...] = jnp.concatenate(
            [h1_r[...] * dis, jnp.zeros((RB, 64), _F32)], axis=1)

    spec128 = pl.BlockSpec((RB, 128), lambda i: (i, 0))
    spec64 = pl.BlockSpec((RB, 64), lambda i: (i, 0))
    return pl.pallas_call(
        body,
        grid=(NB,),
        in_specs=[spec128, spec128, spec128, spec128, spec64,
                  pl.BlockSpec((1, 64), lambda i: (0, 0))],
        out_specs=[spec128, spec64],
        out_shape=[_sds((N, 128)), _sds((N, 64))],
    )(s1, s2, aaug, baug, h1, bea)


def _tc_gcn1(m1a, m1b, g1, agg, s2, wg2, bg1):
    def body(m1a_r, m1b_r, g1_r, agg_r, s2_r, wg2_r, bg1_r, g2_o):
        co = s2_r[...][:, 64:65]
        dis = lax.rsqrt(1.0 + co)
        m1 = (m1a_r[...] + m1b_r[...])[:, :64]
        g1v = g1_r[...][:, :64]
        h = jnp.maximum(dis * (m1 + g1v) + bg1_r[...], 0.0) * agg_r[...]
        g2_o[...] = _dot(h, wg2_r[...]) * dis

    spec128 = pl.BlockSpec((RB, 128), lambda i: (i, 0))
    spec64 = pl.BlockSpec((RB, 64), lambda i: (i, 0))
    return pl.pallas_call(
        body,
        grid=(NB,),
        in_specs=[spec128, spec128, spec128, spec64, spec128,
                  pl.BlockSpec((64, 128), lambda i: (0, 0)),
                  pl.BlockSpec((1, 64), lambda i: (0, 0))],
        out_specs=pl.BlockSpec((RB, 128), lambda i: (i, 0)),
        out_shape=_sds((N, 128)),
    )(m1a, m1b, g1, agg, s2, wg2, bg1)


def _tc_gcn2_pack(m2a, m2b, g2, s2, u, v, wna, bna, wpq, bg2):
    def body(m2a_r, m2b_r, g2_r, s2_r, u_r, v_r,
             wna_r, bna_r, wpq_r, bg2_r, upi_o, vqi_o, h2_o):
        co = s2_r[...][:, 64:65]
        dis = lax.rsqrt(1.0 + co)
        h2 = jnp.maximum(dis * (m2a_r[...] + m2b_r[...] + g2_r[...]) + bg2_r[...], 0.0)
        ni = jnp.maximum(_dot(h2, wna_r[...]) + bna_r[...], 0.0)
        pq = _dot(ni, wpq_r[...])
        upi_o[...] = _pack2(u_r[...], pq[:, :128])
        vqi_o[...] = _pack2(v_r[...], pq[:, 128:])
        h2_o[...] = h2

    spec128 = pl.BlockSpec((RB, 128), lambda i: (i, 0))
    return pl.pallas_call(
        body,
        grid=(NB,),
        in_specs=[spec128, spec128, spec128, spec128, spec128, spec128,
                  pl.BlockSpec((128, 128), lambda i: (0, 0)),
                  pl.BlockSpec((1, 128), lambda i: (0, 0)),
                  pl.BlockSpec((128, 256), lambda i: (0, 0)),
                  pl.BlockSpec((1, 128), lambda i: (0, 0))],
        out_specs=[spec128, spec128, spec128],
        out_shape=[_sds((N, 128), jnp.int32), _sds((N, 128), jnp.int32),
                   _sds((N, 128))],
    )(m2a, m2b, g2, s2, u, v, wna, bna, wpq, bg2)


def _tc_node_head(h2, wn1k, beta_n, wn2):
    def body(h2_r, wn1k_r, bn_r, wn2_r, nl_o):
        nx = _dot(h2_r[...], wn1k_r[...]) + bn_r[...]
        nx = jnp.where(nx > 0, nx, 0.2 * nx)
        logits = _dot(nx, wn2_r[...])
        logits = logits - jnp.max(logits, axis=1, keepdims=True)
        el = jnp.exp(logits)
        nl_o[...] = el / jnp.sum(el, axis=1, keepdims=True)

    return pl.pallas_call(
        body,
        grid=(NB,),
        in_specs=[pl.BlockSpec((RB, 128), lambda i: (i, 0)),
                  pl.BlockSpec((128, 64), lambda i: (0, 0)),
                  pl.BlockSpec((1, 64), lambda i: (0, 0)),
                  pl.BlockSpec((64, 160), lambda i: (0, 0))],
        out_specs=pl.BlockSpec((RB, 160), lambda i: (i, 0)),
        out_shape=_sds((N, 160)),
    )(h2, wn1k, beta_n, wn2)


def _tc_edge_head(g1i, g2i, wm2, bm2, we1k, beta_e, we2p, bm1, bnir,
                  m_init, s_init, he):
    neb = he // EBLK

    def body(g1_r, g2_r, wm2_r, bm2_r, we1_r, be_r, we2_r, bm1_r, bnir_r,
             mi_r, si_r, z_o, m_o, s_o):
        i = pl.program_id(0)
        w1 = g1_r[...]
        w2 = g2_r[...]
        t_pre = _unpack_hi(w1) + _unpack_hi(w2) + bm1_r[...]
        a_pre = _unpack_lo(w1) + _unpack_lo(w2) + bnir_r[...]
        t = jnp.maximum(t_pre, 0.0) * _sigmoid(a_pre)
        ef = jnp.maximum(_dot(t, wm2_r[...]) + bm2_r[...], 0.0)
        ex = _dot(ef, we1_r[...]) + be_r[...]
        ex = jnp.where(ex > 0, ex, 0.2 * ex)
        z = _dot(ex, we2_r[...])
        z_o[...] = z

        @pl.when(i == 0)
        def _():
            m_o[...] = mi_r[...]
            s_o[...] = si_r[...]

        bmax = jnp.max(z, axis=0, keepdims=True)
        m_old = m_o[0:1, :]
        s_old = s_o[0:1, :]
        m_new = jnp.maximum(m_old, bmax)
        s_new = s_old * jnp.exp(m_old - m_new) + jnp.sum(
            jnp.exp(z - m_new), axis=0, keepdims=True)
        m_o[...] = jnp.broadcast_to(m_new, (8, 32))
        s_o[...] = jnp.broadcast_to(s_new, (8, 32))

    speci = pl.BlockSpec((EBLK, 128), lambda i: (i, 0))
    spec_ms = pl.BlockSpec((8, 32), lambda i: (0, 0))
    return pl.pallas_call(
        body,
        grid=(neb,),
        in_specs=[speci, speci,
                  pl.BlockSpec((128, 256), lambda i: (0, 0)),
                  pl.BlockSpec((1, 256), lambda i: (0, 0)),
                  pl.BlockSpec((256, 128), lambda i: (0, 0)),
                  pl.BlockSpec((1, 128), lambda i: (0, 0)),
                  pl.BlockSpec((128, 32), lambda i: (0, 0)),
                  pl.BlockSpec((1, 128), lambda i: (0, 0)),
                  pl.BlockSpec((1, 128), lambda i: (0, 0)),
                  spec_ms, spec_ms],
        out_specs=[pl.BlockSpec((EBLK, 32), lambda i: (i, 0)),
                   spec_ms, spec_ms],
        out_shape=[_sds((he, 32)), _sds((8, 32)), _sds((8, 32))],
    )(g1i, g2i, wm2, bm2, we1k, beta_e, we2p, bm1, bnir, m_init, s_init)


def _tc_edge_softmax(z, m, s, he):
    sblk = EBLK

    def body(z_r, m_r, s_r, out_o):
        val = jnp.exp(z_r[...] - m_r[0:1, :]) / s_r[0:1, :]
        out_o[...] = val[:, :27]

    return pl.pallas_call(
        body,
        grid=(he // sblk,),
        in_specs=[pl.BlockSpec((sblk, 32), lambda i: (i, 0)),
                  pl.BlockSpec((8, 32), lambda i: (0, 0)),
                  pl.BlockSpec((8, 32), lambda i: (0, 0))],
        out_specs=pl.BlockSpec((sblk, 27), lambda i: (i, 0)),
        out_shape=_sds((he, 27)),
    )(z, m, s)


def kernel(node_feats, edge_index, Wg1, bg1, Wg2, bg2, Wea, bea, Wna, bna,
           Wnir, bnir, Wm1, bm1, Wm2, bm2, Wn1, gamma_n, beta_n, Wn2,
           We1, gamma_e, beta_e, We2):
    x = node_feats
    subj = edge_index[:, 0]
    obj = edge_index[:, 1]

    wa = Wea[:128] - Wea[128:]
    wb = Wea[128:]
    wu = Wm1[:128] - Wm1[128:]
    wv = Wm1[128:]
    wpq = jnp.concatenate([Wnir[:128], Wnir[128:]], axis=1)
    kn = gamma_n / jnp.sqrt(1.0 + 1e-5)
    wn1k = Wn1 * kn[None, :]
    ke = gamma_e / jnp.sqrt(1.0 + 1e-5)
    we1k = We1 * ke[None, :]
    we2p = jnp.concatenate([We2, jnp.zeros((128, 5), _F32)], axis=1)
    r = lambda v: v.reshape(1, -1)
    zeros128 = jnp.zeros((SLAB, 128), _F32)

    aaug, baug, h1, u, v = _tc_proj(x, wa, wb, Wg1, wu, wv)
    s1, s2 = _sc_attn_pair(baug, aaug, subj, obj, zeros128)
    g1, agg = _tc_attn_combine(s1, s2, aaug, baug, h1, r(bea))
    m1p = _sc_segment_sum(g1, subj, obj, zeros128)
    g2 = _tc_gcn1(m1p[0], m1p[1], g1, agg, s2, Wg2, r(bg1))
    m2p = _sc_segment_sum(g2, subj, obj, zeros128)
    upi, vqi, h2 = _tc_gcn2_pack(m2p[0], m2p[1], g2, s2, u, v,
                                 Wna, r(bna), wpq, r(bg2))
    m_init = jnp.full((8, 32), -1e30, _F32)
    s_init = jnp.zeros((8, 32), _F32)
    g1a, g2a = _sc_edge_gather(upi, vqi, subj, obj, 0, HEA)
    g1b, g2b = _sc_edge_gather(upi, vqi, subj, obj, HEA, HEB)
    za, ma, sa = _tc_edge_head(g1a, g2a, Wm2, r(bm2), we1k, r(beta_e),
                               we2p, r(bm1), r(bnir), m_init, s_init, HEA)
    node_logits = _tc_node_head(h2, wn1k, r(beta_n), Wn2)
    zb, m, s = _tc_edge_head(g1b, g2b, Wm2, r(bm2), we1k, r(beta_e),
                             we2p, r(bm1), r(bnir), ma, sa, HEB)
    ea = _tc_edge_softmax(za, m, s, HEA)
    eb = _tc_edge_softmax(zb, m, s, HEB)
    edge_logits = jnp.concatenate([ea, eb], axis=0)
    return node_logits, edge_logits


# speedup vs baseline: 9.9132x; 1.0214x over previous
import functools

import jax
import jax.numpy as jnp
from jax import lax
from jax.experimental import pallas as pl
from jax.experimental.pallas import tpu as pltpu
from jax.experimental.pallas import tpu_sc as plsc

N = 10000
E = 320000
NC = 2
NS = 16
NW = NC * NS
EW = E // NW
CH = 80
NCHUNK = EW // CH
SLOTS = 5
NGRP = NCHUNK // SLOTS
CHS = 40
NCHS = EW // CHS
NGRPS = NCHS // SLOTS
EW2 = E // NS
NCHS2 = EW2 // CHS
NGRPS2 = NCHS2 // SLOTS
NPAD = 10240
SLAB = NPAD // NS

RB = 2000
NB = N // RB
HEA = 166400
HEB = E - HEA
EBLK = 3200

_F32 = jnp.float32
_MASK_HI = -65536


def _sds(shape, dtype=_F32):
    return jax.ShapeDtypeStruct(shape, dtype)


def _sigmoid(x):
    return 1.0 / (1.0 + jnp.exp(-x))


def _bf16_bits(x):
    b = lax.bitcast_convert_type(x, jnp.int32)
    return b + 0x7FFF + (lax.shift_right_logical(b, 16) & 1)


def _pack2(hi, lo):
    return (_bf16_bits(hi) & _MASK_HI) | lax.shift_right_logical(_bf16_bits(lo), 16)


def _unpack_hi(w):
    return lax.bitcast_convert_type(w & _MASK_HI, _F32)


def _unpack_lo(w):
    return lax.bitcast_convert_type(jnp.left_shift(w, 16), _F32)


def _mesh():
    return plsc.VectorSubcoreMesh(
        core_axis_name="c", subcore_axis_name="s", num_cores=NC, num_subcores=NS)


def _sc_attn_pair(baug, aaug, subj, obj, zeros):
    @functools.partial(
        pl.kernel,
        out_type=[_sds((NPAD, 128)), _sds((NPAD, 128))],
        mesh=_mesh(),
        scratch_types=[
            pltpu.VMEM((EW2,), jnp.int32),
            pltpu.VMEM((SLOTS, CHS), jnp.int32),
            pltpu.VMEM((SLOTS, CHS, 128), _F32),
            pltpu.VMEM_SHARED((NPAD, 128), _F32),
        ] + [pltpu.SemaphoreType.DMA] * (2 * SLOTS),
    )
    def body(b_hbm, a_hbm, s_hbm, o_hbm, z_hbm, s1_hbm, s2_hbm,
             gflat, sidx, rows, acc, *sems):
        gsem = sems[:SLOTS]
        ssem = sems[SLOTS:]
        cid = lax.axis_index("c")
        sid = lax.axis_index("s")
        base = sid * EW2
        pltpu.sync_copy(z_hbm, acc.at[pl.ds(sid * SLAB, SLAB), :])

        def run(tab_hbm, gsrc_hbm, ssrc_hbm, out_hbm):
            pltpu.sync_copy(gsrc_hbm.at[pl.ds(base, EW2)], gflat)
            plsc.subcore_barrier()

            def group(g, carry):
                c0 = g * SLOTS
                descs = []
                for k in range(SLOTS):
                    @pl.when(g > 0)
                    def _(k=k):
                        pltpu.make_async_copy(
                            rows.at[k], acc.at[sidx.at[k]], ssem[k]).wait()
                    pltpu.sync_copy(
                        ssrc_hbm.at[pl.ds(base + (c0 + k) * CHS, CHS)],
                        sidx.at[k])
                    descs.append(
                        pltpu.async_copy(
                            tab_hbm.at[gflat.at[pl.ds((c0 + k) * CHS, CHS)]],
                            rows.at[k], gsem[k]))
                for k in range(SLOTS):
                    descs[k].wait()
                    pltpu.async_copy(rows.at[k], acc.at[sidx.at[k]], ssem[k],
                                     add=True)
                return carry

            lax.fori_loop(0, NGRPS2, group, 0)
            for k in range(SLOTS):
                pltpu.make_async_copy(
                    rows.at[k], acc.at[sidx.at[k]], ssem[k]).wait()
            plsc.subcore_barrier()
            sl = pl.ds(sid * SLAB, SLAB)
            pltpu.sync_copy(acc.at[sl, :], out_hbm.at[sl, :])

        @pl.when(cid == 0)
        def _():
            run(b_hbm, o_hbm, s_hbm, s1_hbm)

        @pl.when(cid == 1)
        def _():
            run(a_hbm, s_hbm, o_hbm, s2_hbm)

    return body(baug, aaug, subj, obj, zeros)


def _sc_segment_sum(table, gather_idx, scatter_idx, zeros):
    @functools.partial(
        pl.kernel,
        out_type=_sds((NC, NPAD, 128)),
        mesh=_mesh(),
        scratch_types=[
            pltpu.VMEM((EW,), jnp.int32),
            pltpu.VMEM((SLOTS, CHS), jnp.int32),
            pltpu.VMEM((SLOTS, CHS, 128), _F32),
            pltpu.VMEM_SHARED((NPAD, 128), _F32),
        ] + [pltpu.SemaphoreType.DMA] * (2 * SLOTS),
    )
    def body(tab_hbm, g_hbm, s_hbm, z_hbm, out_hbm, gflat, sidx, rows, acc, *sems):
        gsem = sems[:SLOTS]
        ssem = sems[SLOTS:]
        cid = lax.axis_index("c")
        sid = lax.axis_index("s")
        wid = cid * NS + sid
        base = wid * EW
        pltpu.sync_copy(z_hbm, acc.at[pl.ds(sid * SLAB, SLAB), :])
        pltpu.sync_copy(g_hbm.at[pl.ds(base, EW)], gflat)
        plsc.subcore_barrier()

        def group(g, carry):
            c0 = g * SLOTS
            descs = []
            for k in range(SLOTS):
                @pl.when(g > 0)
                def _(k=k):
                    pltpu.make_async_copy(
                        rows.at[k], acc.at[sidx.at[k]], ssem[k]).wait()
                pltpu.sync_copy(
                    s_hbm.at[pl.ds(base + (c0 + k) * CHS, CHS)], sidx.at[k])
                descs.append(
                    pltpu.async_copy(
                        tab_hbm.at[gflat.at[pl.ds((c0 + k) * CHS, CHS)]],
                        rows.at[k], gsem[k]))
            for k in range(SLOTS):
                descs[k].wait()
                pltpu.async_copy(rows.at[k], acc.at[sidx.at[k]], ssem[k],
                                 add=True)
            return carry

        lax.fori_loop(0, NGRPS, group, 0)
        for k in range(SLOTS):
            pltpu.make_async_copy(rows.at[k], acc.at[sidx.at[k]], ssem[k]).wait()
        plsc.subcore_barrier()
        sl = pl.ds(sid * SLAB, SLAB)
        pltpu.sync_copy(acc.at[sl, :], out_hbm.at[cid, sl, :])

    return body(table, gather_idx, scatter_idx, zeros)


def _sc_edge_gather(upi, vqi, subj, obj, start, he):
    ewh = he // NW
    nch = ewh // CH
    ngrp = nch // SLOTS

    @functools.partial(
        pl.kernel,
        out_type=[_sds((he, 128), jnp.int32), _sds((he, 128), jnp.int32)],
        mesh=_mesh(),
        scratch_types=[
            pltpu.VMEM((ewh,), jnp.int32),
            pltpu.VMEM((ewh,), jnp.int32),
            pltpu.VMEM((SLOTS, CH, 128), jnp.int32),
            pltpu.VMEM((SLOTS, CH, 128), jnp.int32),
        ] + [pltpu.SemaphoreType.DMA] * (4 * SLOTS),
    )
    def body(up_hbm, vq_hbm, s_hbm, o_hbm, g1_hbm, g2_hbm,
             sflat, oflat, r1, r2, *sems):
        gs1 = sems[:SLOTS]
        gs2 = sems[SLOTS:2 * SLOTS]
        ws1 = sems[2 * SLOTS:3 * SLOTS]
        ws2 = sems[3 * SLOTS:]
        cid = lax.axis_index("c")
        sid = lax.axis_index("s")
        wid = cid * NS + sid
        base = wid * ewh
        pltpu.sync_copy(s_hbm.at[pl.ds(start + base, ewh)], sflat)
        pltpu.sync_copy(o_hbm.at[pl.ds(start + base, ewh)], oflat)

        def group(g, carry):
            c0 = g * SLOTS
            descs = []
            for k in range(SLOTS):
                @pl.when(g > 0)
                def _(k=k, c0=c0):
                    offp = base + (c0 - SLOTS + k) * CH
                    pltpu.make_async_copy(
                        r1.at[k], g1_hbm.at[pl.ds(offp, CH), :], ws1[k]).wait()
                    pltpu.make_async_copy(
                        r2.at[k], g2_hbm.at[pl.ds(offp, CH), :], ws2[k]).wait()
                sl = pl.ds((c0 + k) * CH, CH)
                descs.append(
                    pltpu.async_copy(up_hbm.at[sflat.at[sl]], r1.at[k], gs1[k]))
                descs.append(
                    pltpu.async_copy(vq_hbm.at[oflat.at[sl]], r2.at[k], gs2[k]))
            for k in range(SLOTS):
                off = base + (c0 + k) * CH
                descs[2 * k].wait()
                pltpu.async_copy(r1.at[k], g1_hbm.at[pl.ds(off, CH), :], ws1[k])
                descs[2 * k + 1].wait()
                pltpu.async_copy(r2.at[k], g2_hbm.at[pl.ds(off, CH), :], ws2[k])
            return carry

        lax.fori_loop(0, ngrp, group, 0)
        for k in range(SLOTS):
            offp = base + ((ngrp - 1) * SLOTS + k) * CH
            pltpu.make_async_copy(
                r1.at[k], g1_hbm.at[pl.ds(offp, CH), :], ws1[k]).wait()
            pltpu.make_async_copy(
                r2.at[k], g2_hbm.at[pl.ds(offp, CH), :], ws2[k]).wait()

    return body(upi, vqi, subj, obj)


def _dot(a, b):
    return jnp.dot(a, b, preferred_element_type=_F32)


def _tc_proj(x, wa, wb, wg1, wu, wv):
    def body(x_r, wa_r, wb_r, wg_r, wu_r, wv_r, aaug_o, baug_o, h1_o, u_o, v_o):
        xb = x_r[...]
        ones = jnp.ones((RB, 64), _F32)
        aaug_o[...] = jnp.concatenate([_dot(xb, wa_r[...]), ones], axis=1)
        baug_o[...] = jnp.concatenate([_dot(xb, wb_r[...]), ones], axis=1)
        h1_o[...] = _dot(xb, wg_r[...])
        u_o[...] = _dot(xb, wu_r[...])
        v_o[...] = _dot(xb, wv_r[...])

    return pl.pallas_call(
        body,
        grid=(NB,),
        in_specs=[
            pl.BlockSpec((RB, 128), lambda i: (i, 0)),
            pl.BlockSpec((128, 64), lambda i: (0, 0)),
            pl.BlockSpec((128, 64), lambda i: (0, 0)),
            pl.BlockSpec((128, 64), lambda i: (0, 0)),
            pl.BlockSpec((128, 128), lambda i: (0, 0)),
            pl.BlockSpec((128, 128), lambda i: (0, 0)),
        ],
        out_specs=[
            pl.BlockSpec((RB, 128), lambda i: (i, 0)),
            pl.BlockSpec((RB, 128), lambda i: (i, 0)),
            pl.BlockSpec((RB, 64), lambda i: (i, 0)),
            pl.BlockSpec((RB, 128), lambda i: (i, 0)),
            pl.BlockSpec((RB, 128), lambda i: (i, 0)),
        ],
        out_shape=[_sds((N, 128)), _sds((N, 128)), _sds((N, 64)),
                   _sds((N, 128)), _sds((N, 128))],
    )(x, wa, wb, wg1, wu, wv)


def _tc_attn_combine(s1, s2, aaug, baug, h1, bea):
    def body(s1_r, s2_r, a_r, b_r, h1_r, bea_r, g1_o, agg_o):
        s1 = s1_r[...]
        s2 = s2_r[...]
        a = a_r[...][:, :64]
        b = b_r[...][:, :64]
        cs = s1[:, 64:65]
        co = s2[:, 64:65]
        bea_v = bea_r[...]
        mean_s = (cs * (a + bea_v) + s1[:, :64]) / jnp.maximum(cs, 1.0)
        mean_o = (s2[:, :64] + co * (b + bea_v)) / jnp.maximum(co, 1.0)
        agg_o[...] = _sigmoid(mean_s * mean_o)
        dis = lax.rsqrt(1.0 + co)
        g1_o[...] = jnp.concatenate(
            [h1_r[...] * dis, jnp.zeros((RB, 64), _F32)], axis=1)

    spec128 = pl.BlockSpec((RB, 128), lambda i: (i, 0))
    spec64 = pl.BlockSpec((RB, 64), lambda i: (i, 0))
    return pl.pallas_call(
        body,
        grid=(NB,),
        in_specs=[spec128, spec128, spec128, spec128, spec64,
                  pl.BlockSpec((1, 64), lambda i: (0, 0))],
        out_specs=[spec128, spec64],
        out_shape=[_sds((N, 128)), _sds((N, 64))],
    )(s1, s2, aaug, baug, h1, bea)


def _tc_gcn1(m1a, m1b, g1, agg, s2, wg2, bg1):
    def body(m1a_r, m1b_r, g1_r, agg_r, s2_r, wg2_r, bg1_r, g2_o):
        co = s2_r[...][:, 64:65]
        dis = lax.rsqrt(1.0 + co)
        m1 = (m1a_r[...] + m1b_r[...])[:, :64]
        g1v = g1_r[...][:, :64]
        h = jnp.maximum(dis * (m1 + g1v) + bg1_r[...], 0.0) * agg_r[...]
        g2_o[...] = _dot(h, wg2_r[...]) * dis

    spec128 = pl.BlockSpec((RB, 128), lambda i: (i, 0))
    spec64 = pl.BlockSpec((RB, 64), lambda i: (i, 0))
    return pl.pallas_call(
        body,
        grid=(NB,),
        in_specs=[spec128, spec128, spec128, spec64, spec128,
                  pl.BlockSpec((64, 128), lambda i: (0, 0)),
                  pl.BlockSpec((1, 64), lambda i: (0, 0))],
        out_specs=pl.BlockSpec((RB, 128), lambda i: (i, 0)),
        out_shape=_sds((N, 128)),
    )(m1a, m1b, g1, agg, s2, wg2, bg1)


def _tc_gcn2_pack(m2a, m2b, g2, s2, u, v, wna, bna, wpq, bg2):
    def body(m2a_r, m2b_r, g2_r, s2_r, u_r, v_r,
             wna_r, bna_r, wpq_r, bg2_r, upi_o, vqi_o, h2_o):
        co = s2_r[...][:, 64:65]
        dis = lax.rsqrt(1.0 + co)
        h2 = jnp.maximum(dis * (m2a_r[...] + m2b_r[...] + g2_r[...]) + bg2_r[...], 0.0)
        ni = jnp.maximum(_dot(h2, wna_r[...]) + bna_r[...], 0.0)
        pq = _dot(ni, wpq_r[...])
        upi_o[...] = _pack2(u_r[...], pq[:, :128])
        vqi_o[...] = _pack2(v_r[...], pq[:, 128:])
        h2_o[...] = h2

    spec128 = pl.BlockSpec((RB, 128), lambda i: (i, 0))
    return pl.pallas_call(
        body,
        grid=(NB,),
        in_specs=[spec128, spec128, spec128, spec128, spec128, spec128,
                  pl.BlockSpec((128, 128), lambda i: (0, 0)),
                  pl.BlockSpec((1, 128), lambda i: (0, 0)),
                  pl.BlockSpec((128, 256), lambda i: (0, 0)),
                  pl.BlockSpec((1, 128), lambda i: (0, 0))],
        out_specs=[spec128, spec128, spec128],
        out_shape=[_sds((N, 128), jnp.int32), _sds((N, 128), jnp.int32),
                   _sds((N, 128))],
    )(m2a, m2b, g2, s2, u, v, wna, bna, wpq, bg2)


def _tc_node_head(h2, wn1k, beta_n, wn2):
    def body(h2_r, wn1k_r, bn_r, wn2_r, nl_o):
        nx = _dot(h2_r[...], wn1k_r[...]) + bn_r[...]
        nx = jnp.where(nx > 0, nx, 0.2 * nx)
        logits = _dot(nx, wn2_r[...])
        logits = logits - jnp.max(logits, axis=1, keepdims=True)
        el = jnp.exp(logits)
        nl_o[...] = el / jnp.sum(el, axis=1, keepdims=True)

    return pl.pallas_call(
        body,
        grid=(NB,),
        in_specs=[pl.BlockSpec((RB, 128), lambda i: (i, 0)),
                  pl.BlockSpec((128, 64), lambda i: (0, 0)),
                  pl.BlockSpec((1, 64), lambda i: (0, 0)),
                  pl.BlockSpec((64, 160), lambda i: (0, 0))],
        out_specs=pl.BlockSpec((RB, 160), lambda i: (i, 0)),
        out_shape=_sds((N, 160)),
    )(h2, wn1k, beta_n, wn2)


def _tc_edge_head(g1i, g2i, wm2, bm2, we1k, beta_e, we2p, bm1, bnir,
                  m_init, s_init, he):
    neb = he // EBLK

    def body(g1_r, g2_r, wm2_r, bm2_r, we1_r, be_r, we2_r, bm1_r, bnir_r,
             mi_r, si_r, z_o, m_o, s_o):
        i = pl.program_id(0)
        w1 = g1_r[...]
        w2 = g2_r[...]
        t_pre = _unpack_hi(w1) + _unpack_hi(w2) + bm1_r[...]
        a_pre = _unpack_lo(w1) + _unpack_lo(w2) + bnir_r[...]
        t = jnp.maximum(t_pre, 0.0) * _sigmoid(a_pre)
        ef = jnp.maximum(_dot(t, wm2_r[...]) + bm2_r[...], 0.0)
        ex = _dot(ef, we1_r[...]) + be_r[...]
        ex = jnp.where(ex > 0, ex, 0.2 * ex)
        z = _dot(ex, we2_r[...])
        z_o[...] = z

        @pl.when(i == 0)
        def _():
            m_o[...] = mi_r[...]
            s_o[...] = si_r[...]

        bmax = jnp.max(z, axis=0, keepdims=True)
        m_old = m_o[0:1, :]
        s_old = s_o[0:1, :]
        m_new = jnp.maximum(m_old, bmax)
        s_new = s_old * jnp.exp(m_old - m_new) + jnp.sum(
            jnp.exp(z - m_new), axis=0, keepdims=True)
        m_o[...] = jnp.broadcast_to(m_new, (8, 32))
        s_o[...] = jnp.broadcast_to(s_new, (8, 32))

    speci = pl.BlockSpec((EBLK, 128), lambda i: (i, 0))
    spec_ms = pl.BlockSpec((8, 32), lambda i: (0, 0))
    return pl.pallas_call(
        body,
        grid=(neb,),
        in_specs=[speci, speci,
                  pl.BlockSpec((128, 256), lambda i: (0, 0)),
                  pl.BlockSpec((1, 256), lambda i: (0, 0)),
                  pl.BlockSpec((256, 128), lambda i: (0, 0)),
                  pl.BlockSpec((1, 128), lambda i: (0, 0)),
                  pl.BlockSpec((128, 32), lambda i: (0, 0)),
                  pl.BlockSpec((1, 128), lambda i: (0, 0)),
                  pl.BlockSpec((1, 128), lambda i: (0, 0)),
                  spec_ms, spec_ms],
        out_specs=[pl.BlockSpec((EBLK, 32), lambda i: (i, 0)),
                   spec_ms, spec_ms],
        out_shape=[_sds((he, 32)), _sds((8, 32)), _sds((8, 32))],
    )(g1i, g2i, wm2, bm2, we1k, beta_e, we2p, bm1, bnir, m_init, s_init)


def _tc_edge_softmax(z, m, s, he):
    sblk = EBLK

    def body(z_r, m_r, s_r, out_o):
        val = jnp.exp(z_r[...] - m_r[0:1, :]) / s_r[0:1, :]
        out_o[...] = val[:, :27]

    return pl.pallas_call(
        body,
        grid=(he // sblk,),
        in_specs=[pl.BlockSpec((sblk, 32), lambda i: (i, 0)),
                  pl.BlockSpec((8, 32), lambda i: (0, 0)),
                  pl.BlockSpec((8, 32), lambda i: (0, 0))],
        out_specs=pl.BlockSpec((sblk, 27), lambda i: (i, 0)),
        out_shape=_sds((he, 27)),
    )(z, m, s)


def kernel(node_feats, edge_index, Wg1, bg1, Wg2, bg2, Wea, bea, Wna, bna,
           Wnir, bnir, Wm1, bm1, Wm2, bm2, Wn1, gamma_n, beta_n, Wn2,
           We1, gamma_e, beta_e, We2):
    x = node_feats
    subj = edge_index[:, 0]
    obj = edge_index[:, 1]

    wa = Wea[:128] - Wea[128:]
    wb = Wea[128:]
    wu = Wm1[:128] - Wm1[128:]
    wv = Wm1[128:]
    wpq = jnp.concatenate([Wnir[:128], Wnir[128:]], axis=1)
    kn = gamma_n / jnp.sqrt(1.0 + 1e-5)
    wn1k = Wn1 * kn[None, :]
    ke = gamma_e / jnp.sqrt(1.0 + 1e-5)
    we1k = We1 * ke[None, :]
    we2p = jnp.concatenate([We2, jnp.zeros((128, 5), _F32)], axis=1)
    r = lambda v: v.reshape(1, -1)
    zeros128 = jnp.zeros((SLAB, 128), _F32)

    aaug, baug, h1, u, v = _tc_proj(x, wa, wb, Wg1, wu, wv)
    s1, s2 = _sc_attn_pair(baug, aaug, subj, obj, zeros128)
    g1, agg = _tc_attn_combine(s1, s2, aaug, baug, h1, r(bea))
    m1p = _sc_segment_sum(g1, subj, obj, zeros128)
    g2 = _tc_gcn1(m1p[0], m1p[1], g1, agg, s2, Wg2, r(bg1))
    m2p = _sc_segment_sum(g2, subj, obj, zeros128)
    upi, vqi, h2 = _tc_gcn2_pack(m2p[0], m2p[1], g2, s2, u, v,
                                 Wna, r(bna), wpq, r(bg2))
    m_init = jnp.full((8, 32), -1e30, _F32)
    s_init = jnp.zeros((8, 32), _F32)
    g1a, g2a = _sc_edge_gather(upi, vqi, subj, obj, 0, HEA)
    g1b, g2b = _sc_edge_gather(upi, vqi, subj, obj, HEA, HEB)
    za, ma, sa = _tc_edge_head(g1a, g2a, Wm2, r(bm2), we1k, r(beta_e),
                               we2p, r(bm1), r(bnir), m_init, s_init, HEA)
    node_logits = _tc_node_head(h2, wn1k, r(beta_n), Wn2)
    zb, m, s = _tc_edge_head(g1b, g2b, Wm2, r(bm2), we1k, r(beta_e),
                             we2p, r(bm1), r(bnir), ma, sa, HEB)
    ea = _tc_edge_softmax(za, m, s, HEA)
    eb = _tc_edge_softmax(zb, m, s, HEB)
    edge_logits = jnp.concatenate([ea, eb], axis=0)
    return node_logits, edge_logits
